# baseline (speedup 1.0000x reference)
import jax
import jax.numpy as jnp
from jax import lax
from jax.experimental import pallas as pl
from jax.experimental.pallas import tpu as pltpu
from jax.experimental.pallas import tpu_sc as plsc

N = 10000
E = 320000
D = 128
DE = 16
G = 16

NC = 2
NS = 16
L = 16

N_PAD = 10240
E_PAD = 327680
EPT = E_PAD // NS
K = 128
CPT = EPT // K
NPASS = 2
HCPT = CPT // NPASS
DH = D // NC
ROWS_PER_TILE = N_PAD // NS

NB = 256
EB = 2000


def _edge_dense_body(ea_ref, we1_ref, ae1_ref, we2_ref, ae2_ref,
                     e1_ref, e2_ref, easum_ref):
    i = pl.program_id(0)
    ea = ea_ref[...]
    e1_ref[...] = ea @ (we1_ref[...] @ ae1_ref[...])
    e2_ref[...] = ea @ (we2_ref[...] @ ae2_ref[...])

    @pl.when(i == 0)
    def _():
        easum_ref[...] = jnp.zeros_like(easum_ref)

    easum_ref[...] += jnp.sum(ea, axis=0, keepdims=True)


def _edge_dense(ea, we1, ae1v, we2, ae2v):
    grid = E // EB
    return pl.pallas_call(
        _edge_dense_body,
        grid=(grid,),
        in_specs=[
            pl.BlockSpec((EB, DE), lambda i: (i, 0)),
            pl.BlockSpec((DE, D), lambda i: (0, 0)),
            pl.BlockSpec((D, 1), lambda i: (0, 0)),
            pl.BlockSpec((DE, D), lambda i: (0, 0)),
            pl.BlockSpec((D, 1), lambda i: (0, 0)),
        ],
        out_specs=[
            pl.BlockSpec((EB, 1), lambda i: (i, 0)),
            pl.BlockSpec((EB, 1), lambda i: (i, 0)),
            pl.BlockSpec((1, DE), lambda i: (0, 0)),
        ],
        out_shape=[
            jax.ShapeDtypeStruct((E, 1), jnp.float32),
            jax.ShapeDtypeStruct((E, 1), jnp.float32),
            jax.ShapeDtypeStruct((1, DE), jnp.float32),
        ],
    )(ea, we1, ae1v, we2, ae2v)


def _node_tail(h, as_ref, ad_ref, we_ref, aev_ref, easum_ref,
               h0_ref, h1_ref, hs_ref, hd_ref, sh_ref):
    h0_ref[...] = h[:, :DH]
    h1_ref[...] = h[:, DH:]
    hs = h @ as_ref[...]
    hd = h @ ad_ref[...]
    w_e = we_ref[...] @ aev_ref[...]
    c = (easum_ref[...] @ w_e)[0, 0] * (1.0 / E)
    t = hs + hd + c
    hs_ref[...] = hs
    hd_ref[...] = hd
    sh_ref[...] = jnp.where(t >= 0.0, t, 0.2 * t)


_NODE_OUT_SPECS = [
    pl.BlockSpec((NB, DH), lambda i: (i, 0)),
    pl.BlockSpec((NB, DH), lambda i: (i, 0)),
    pl.BlockSpec((NB, 1), lambda i: (i, 0)),
    pl.BlockSpec((NB, 1), lambda i: (i, 0)),
    pl.BlockSpec((NB, 1), lambda i: (i, 0)),
]

_NODE_OUT_SHAPE = [
    jax.ShapeDtypeStruct((N_PAD, DH), jnp.float32),
    jax.ShapeDtypeStruct((N_PAD, DH), jnp.float32),
    jax.ShapeDtypeStruct((N_PAD, 1), jnp.float32),
    jax.ShapeDtypeStruct((N_PAD, 1), jnp.float32),
    jax.ShapeDtypeStruct((N_PAD, 1), jnp.float32),
]


def _node_dense_body(x_ref, w_ref, as_ref, ad_ref, we_ref, aev_ref, easum_ref,
                     h0_ref, h1_ref, hs_ref, hd_ref, sh_ref):
    h = x_ref[...] @ w_ref[...]
    _node_tail(h, as_ref, ad_ref, we_ref, aev_ref, easum_ref,
               h0_ref, h1_ref, hs_ref, hd_ref, sh_ref)


def _node_dense(xp, w, asv, adv, we, aev, easum):
    grid = N_PAD // NB
    return pl.pallas_call(
        _node_dense_body,
        grid=(grid,),
        in_specs=[
            pl.BlockSpec((NB, D), lambda i: (i, 0)),
            pl.BlockSpec((D, D), lambda i: (0, 0)),
            pl.BlockSpec((D, 1), lambda i: (0, 0)),
            pl.BlockSpec((D, 1), lambda i: (0, 0)),
            pl.BlockSpec((DE, D), lambda i: (0, 0)),
            pl.BlockSpec((D, 1), lambda i: (0, 0)),
            pl.BlockSpec((1, DE), lambda i: (0, 0)),
        ],
        out_specs=_NODE_OUT_SPECS,
        out_shape=_NODE_OUT_SHAPE,
    )(xp, w, asv, adv, we, aev, easum)


def _mid_dense_body(p0_ref, p1_ref, h0_ref, h1_ref, d_ref, b_ref,
                    w_ref, as_ref, ad_ref, we_ref, aev_ref, easum_ref,
                    h0o_ref, h1o_ref, hs_ref, hd_ref, sh_ref):
    den = d_ref[...] + 1.0
    msg = jnp.concatenate(
        [p0_ref[...] + h0_ref[...], p1_ref[...] + h1_ref[...]], axis=-1)
    y = msg / den + b_ref[...]
    y = y * (1.0 / (1.0 + jnp.exp(-y)))
    h = y @ w_ref[...]
    _node_tail(h, as_ref, ad_ref, we_ref, aev_ref, easum_ref,
               h0o_ref, h1o_ref, hs_ref, hd_ref, sh_ref)


def _mid_dense(p0, p1, h0, h1, d, b2d, w, asv, adv, we, aev, easum):
    grid = N_PAD // NB
    return pl.pallas_call(
        _mid_dense_body,
        grid=(grid,),
        in_specs=[
            pl.BlockSpec((NB, DH), lambda i: (i, 0)),
            pl.BlockSpec((NB, DH), lambda i: (i, 0)),
            pl.BlockSpec((NB, DH), lambda i: (i, 0)),
            pl.BlockSpec((NB, DH), lambda i: (i, 0)),
            pl.BlockSpec((NB, 1), lambda i: (i, 0)),
            pl.BlockSpec((1, D), lambda i: (0, 0)),
            pl.BlockSpec((D, D), lambda i: (0, 0)),
            pl.BlockSpec((D, 1), lambda i: (0, 0)),
            pl.BlockSpec((D, 1), lambda i: (0, 0)),
            pl.BlockSpec((DE, D), lambda i: (0, 0)),
            pl.BlockSpec((D, 1), lambda i: (0, 0)),
            pl.BlockSpec((1, DE), lambda i: (0, 0)),
        ],
        out_specs=_NODE_OUT_SPECS,
        out_shape=_NODE_OUT_SHAPE,
    )(p0, p1, h0, h1, d, b2d, w, asv, adv, we, aev, easum)


def _sc_layer_body(hs_hbm, hd_hbm, sh_hbm, eatt_hbm, src_hbm, dst_hbm,
                   h0_hbm, h1_hbm, out_hbm, den_hbm,
                   hs_v, hd_v, sh_v, eatt_v, src_v, dst_v,
                   ex0, ex1, rows0, rows1,
                   out_sh, den_sh,
                   gsem0, gsem1, ssem0, ssem1, dsem0, dsem1):
    c = lax.axis_index("c")
    s = lax.axis_index("s")
    cbase = s * CPT
    rbufs = (rows0, rows1)
    exbufs = (ex0, ex1)
    gsems = (gsem0, gsem1)
    ssems = (ssem0, ssem1)
    dsems = (dsem0, dsem1)

    pltpu.sync_copy(hs_hbm, hs_v)
    pltpu.sync_copy(hd_hbm, hd_v)
    pltpu.sync_copy(sh_hbm, sh_v)

    def _zrow(i, _):
        for k in range(DH // L):
            rows0[i, pl.ds(k * L, L)] = jnp.zeros((L,), jnp.float32)
        return 0

    lax.fori_loop(0, K, _zrow, 0)
    rbase = s * ROWS_PER_TILE
    for t in range(ROWS_PER_TILE // K):
        pltpu.sync_copy(rows0, out_sh.at[pl.ds(rbase + t * K, K)])

    @pl.when(c == 0)
    def _():
        for t in range(ROWS_PER_TILE // DH):
            pltpu.sync_copy(rows0.at[0],
                            den_sh.at[pl.ds(rbase + t * DH, DH)])

    plsc.subcore_barrier()

    def _issue_gather(j, buf, sem):
        @pl.when(c == 0)
        def _():
            pltpu.async_copy(h0_hbm.at[src_v.at[j]], buf, sem)

        @pl.when(c == 1)
        def _():
            pltpu.async_copy(h1_hbm.at[src_v.at[j]], buf, sem)

    def _chunk(j, b):
        nb = 1 - b
        buf = rbufs[b]
        exb = exbufs[b]

        @pl.when(j >= 1)
        def _():
            pltpu.make_async_copy(rbufs[nb], out_sh.at[dst_v.at[0]],
                                  ssems[nb]).wait()

        @pl.when(j + 1 < HCPT)
        def _():
            _issue_gather(j + 1, rbufs[nb], gsems[nb])

        @pl.when(jnp.logical_and(c == 0, j >= 2))
        def _():
            pltpu.make_async_copy(exb, den_sh.at[dst_v.at[0]],
                                  dsems[b]).wait()

        for k in range(K // L):
            sl = pl.ds(k * L, L)
            isrc = src_v[j, sl]
            idst = dst_v[j, sl]
            av = plsc.load_gather(hs_v, [isrc])
            bv = plsc.load_gather(hd_v, [idst])
            shv = plsc.load_gather(sh_v, [idst])
            al = av + bv + eatt_v[j, sl]
            al = jnp.where(al >= 0.0, al, 0.2 * al)
            exb[sl] = jnp.exp(jnp.minimum(al - shv, 80.0))

        @pl.when(c == 0)
        def _():
            pltpu.async_copy(exb, den_sh.at[dst_v.at[j]], dsems[b], add=True)

        pltpu.make_async_copy(h0_hbm.at[src_v.at[0]], buf, gsems[b]).wait()

        def _scale(v, _):
            base = v * L
            for e in range(L):
                sv = plsc.load_gather(exb, [jnp.full((L,), base + e,
                                                     jnp.int32)])
                for k in range(DH // L):
                    sl = pl.ds(k * L, L)
                    buf[base + e, sl] = buf[base + e, sl] * sv
            return 0

        lax.fori_loop(0, K // L, _scale, 0)
        pltpu.async_copy(buf, out_sh.at[dst_v.at[j]], ssems[b], add=True)

    def _pair(i, _):
        _chunk(i * 2, 0)
        _chunk(i * 2 + 1, 1)
        return 0

    for p in range(NPASS):
        pbase = cbase + p * HCPT
        pltpu.sync_copy(eatt_hbm.at[pl.ds(pbase, HCPT)], eatt_v)
        pltpu.sync_copy(src_hbm.at[pl.ds(pbase, HCPT)], src_v)
        pltpu.sync_copy(dst_hbm.at[pl.ds(pbase, HCPT)], dst_v)
        _issue_gather(0, rows0, gsem0)
        lax.fori_loop(0, HCPT // 2, _pair, 0)
        pltpu.make_async_copy(rows1, out_sh.at[dst_v.at[0]], ssem1).wait()

        @pl.when(c == 0)
        def _():
            pltpu.make_async_copy(ex0, den_sh.at[dst_v.at[0]], dsem0).wait()
            pltpu.make_async_copy(ex1, den_sh.at[dst_v.at[0]], dsem1).wait()

    plsc.subcore_barrier()

    pltpu.sync_copy(out_sh.at[pl.ds(rbase, ROWS_PER_TILE)],
                    out_hbm.at[c, pl.ds(rbase, ROWS_PER_TILE)])

    @pl.when(c == 0)
    def _():
        pltpu.sync_copy(den_sh.at[pl.ds(rbase, ROWS_PER_TILE)],
                        den_hbm.at[pl.ds(rbase, ROWS_PER_TILE)])


_sc_layer = pl.kernel(
    _sc_layer_body,
    out_type=[
        jax.ShapeDtypeStruct((NC, N_PAD, DH), jnp.float32),
        jax.ShapeDtypeStruct((N_PAD,), jnp.float32),
    ],
    mesh=plsc.VectorSubcoreMesh(core_axis_name="c", subcore_axis_name="s"),
    compiler_params=pltpu.CompilerParams(
        use_tc_tiling_on_sc=False, needs_layout_passes=False),
    scratch_types=[
        pltpu.VMEM((N_PAD,), jnp.float32),
        pltpu.VMEM((N_PAD,), jnp.float32),
        pltpu.VMEM((N_PAD,), jnp.float32),
        pltpu.VMEM((HCPT, K), jnp.float32),
        pltpu.VMEM((HCPT, K), jnp.int32),
        pltpu.VMEM((HCPT, K), jnp.int32),
        pltpu.VMEM((K,), jnp.float32),
        pltpu.VMEM((K,), jnp.float32),
        pltpu.VMEM((K, DH), jnp.float32),
        pltpu.VMEM((K, DH), jnp.float32),
        pltpu.VMEM_SHARED((N_PAD, DH), jnp.float32),
        pltpu.VMEM_SHARED((N_PAD,), jnp.float32),
        pltpu.SemaphoreType.DMA,
        pltpu.SemaphoreType.DMA,
        pltpu.SemaphoreType.DMA,
        pltpu.SemaphoreType.DMA,
        pltpu.SemaphoreType.DMA,
        pltpu.SemaphoreType.DMA,
    ],
)


def _epilogue_pool_body(p0_ref, p1_ref, h0_ref, h1_ref, d_ref, b_ref, bid_ref,
                        pooled_ref, cnt_ref):
    i = pl.program_id(0)

    @pl.when(i == 0)
    def _():
        pooled_ref[...] = jnp.zeros_like(pooled_ref)
        cnt_ref[...] = jnp.zeros_like(cnt_ref)

    den = d_ref[...] + 1.0
    msg = jnp.concatenate(
        [p0_ref[...] + h0_ref[...], p1_ref[...] + h1_ref[...]], axis=-1)
    y = msg / den + b_ref[...]
    y = y * (1.0 / (1.0 + jnp.exp(-y)))
    bid = bid_ref[...]
    ones = jnp.ones_like(y)
    for g in range(G):
        m = bid == g
        pooled_ref[g:g + 1, :] += jnp.sum(jnp.where(m, y, 0.0), axis=0,
                                          keepdims=True)
        cnt_ref[g:g + 1, :] += jnp.sum(jnp.where(m, ones, 0.0), axis=0,
                                       keepdims=True)

    @pl.when(i == pl.num_programs(0) - 1)
    def _():
        pooled_ref[...] = pooled_ref[...] / jnp.maximum(cnt_ref[...], 1.0)


def _epilogue_pool(p0, p1, h0, h1, d, b2d, bid2d):
    grid = N_PAD // NB
    return pl.pallas_call(
        _epilogue_pool_body,
        grid=(grid,),
        in_specs=[
            pl.BlockSpec((NB, DH), lambda i: (i, 0)),
            pl.BlockSpec((NB, DH), lambda i: (i, 0)),
            pl.BlockSpec((NB, DH), lambda i: (i, 0)),
            pl.BlockSpec((NB, DH), lambda i: (i, 0)),
            pl.BlockSpec((NB, 1), lambda i: (i, 0)),
            pl.BlockSpec((1, D), lambda i: (0, 0)),
            pl.BlockSpec((NB, 1), lambda i: (i, 0)),
        ],
        out_specs=pl.BlockSpec((G, D), lambda i: (0, 0)),
        out_shape=jax.ShapeDtypeStruct((G, D), jnp.float32),
        scratch_shapes=[pltpu.VMEM((G, D), jnp.float32)],
    )(p0, p1, h0, h1, d, b2d, bid2d)


def kernel(x, edge_index, edge_attr, batch,
           W1, as1, ad1, We1, ae1, b1, W2, as2, ad2, We2, ae2, b2):
    f32 = jnp.float32
    xp = jnp.zeros((N_PAD, D), f32).at[:N].set(x)
    pad_idx = jnp.full((E_PAD - E,), N_PAD - 1, jnp.int32)
    src2d = jnp.concatenate([edge_index[0], pad_idx]).reshape(E_PAD // K, K)
    dst2d = jnp.concatenate([edge_index[1], pad_idx]).reshape(E_PAD // K, K)
    bid2d = jnp.full((N_PAD, 1), G, jnp.int32).at[:N, 0].set(batch)

    eatt1, eatt2, easum = _edge_dense(edge_attr, We1, ae1.reshape(D, 1),
                                      We2, ae2.reshape(D, 1))
    zpad = jnp.zeros((E_PAD // K - E // K, K), f32)
    e1_2d = jnp.concatenate([eatt1.reshape(E // K, K), zpad], axis=0)
    e2_2d = jnp.concatenate([eatt2.reshape(E // K, K), zpad], axis=0)

    h0, h1, hs, hd, sh = _node_dense(xp, W1, as1.reshape(D, 1),
                                     ad1.reshape(D, 1), We1,
                                     ae1.reshape(D, 1), easum)
    parts1, den1 = _sc_layer(hs.reshape(N_PAD), hd.reshape(N_PAD),
                             sh.reshape(N_PAD), e1_2d, src2d, dst2d, h0, h1)
    g0, g1, hs2, hd2, sh2 = _mid_dense(parts1[0], parts1[1], h0, h1,
                                       den1.reshape(N_PAD, 1),
                                       b1.reshape(1, D), W2,
                                       as2.reshape(D, 1), ad2.reshape(D, 1),
                                       We2, ae2.reshape(D, 1), easum)
    parts2, den2 = _sc_layer(hs2.reshape(N_PAD), hd2.reshape(N_PAD),
                             sh2.reshape(N_PAD), e2_2d, src2d, dst2d, g0, g1)
    pooled = _epilogue_pool(parts2[0], parts2[1], g0, g1,
                            den2.reshape(N_PAD, 1), b2.reshape(1, D), bid2d)
    return pooled


# speedup vs baseline: 1.0281x; 1.0281x over previous
import jax
import jax.numpy as jnp
from jax import lax
from jax.experimental import pallas as pl
from jax.experimental.pallas import tpu as pltpu
from jax.experimental.pallas import tpu_sc as plsc

N = 10000
E = 320000
D = 128
DE = 16
G = 16

NC = 2
NS = 16
L = 16

N_PAD = 10240
E_PAD = 327680
EPT = E_PAD // NS
K = 128
CPT = EPT // K
NPASS = 2
HCPT = CPT // NPASS
DH = D // NC
ROWS_PER_TILE = N_PAD // NS

NB = 256
EB = 2000


def _edge_dense_body(ea_ref, we1_ref, ae1_ref, we2_ref, ae2_ref,
                     e1_ref, e2_ref, easum_ref):
    i = pl.program_id(0)
    ea = ea_ref[...]
    e1_ref[...] = ea @ (we1_ref[...] @ ae1_ref[...])
    e2_ref[...] = ea @ (we2_ref[...] @ ae2_ref[...])

    @pl.when(i == 0)
    def _():
        easum_ref[...] = jnp.zeros_like(easum_ref)

    easum_ref[...] += jnp.sum(ea, axis=0, keepdims=True)


def _edge_dense(ea, we1, ae1v, we2, ae2v):
    grid = E // EB
    return pl.pallas_call(
        _edge_dense_body,
        grid=(grid,),
        in_specs=[
            pl.BlockSpec((EB, DE), lambda i: (i, 0)),
            pl.BlockSpec((DE, D), lambda i: (0, 0)),
            pl.BlockSpec((D, 1), lambda i: (0, 0)),
            pl.BlockSpec((DE, D), lambda i: (0, 0)),
            pl.BlockSpec((D, 1), lambda i: (0, 0)),
        ],
        out_specs=[
            pl.BlockSpec((EB, 1), lambda i: (i, 0)),
            pl.BlockSpec((EB, 1), lambda i: (i, 0)),
            pl.BlockSpec((1, DE), lambda i: (0, 0)),
        ],
        out_shape=[
            jax.ShapeDtypeStruct((E, 1), jnp.float32),
            jax.ShapeDtypeStruct((E, 1), jnp.float32),
            jax.ShapeDtypeStruct((1, DE), jnp.float32),
        ],
    )(ea, we1, ae1v, we2, ae2v)


def _node_tail(h, as_ref, ad_ref, we_ref, aev_ref, easum_ref,
               h0_ref, h1_ref, hs_ref, hd_ref, sh_ref):
    h0_ref[...] = h[:, :DH]
    h1_ref[...] = h[:, DH:]
    hs = h @ as_ref[...]
    hd = h @ ad_ref[...]
    w_e = we_ref[...] @ aev_ref[...]
    c = (easum_ref[...] @ w_e)[0, 0] * (1.0 / E)
    t = hs + hd + c
    hs_ref[...] = hs
    hd_ref[...] = hd
    sh_ref[...] = jnp.where(t >= 0.0, t, 0.2 * t)


_NODE_OUT_SPECS = [
    pl.BlockSpec((NB, DH), lambda i: (i, 0)),
    pl.BlockSpec((NB, DH), lambda i: (i, 0)),
    pl.BlockSpec((NB, 1), lambda i: (i, 0)),
    pl.BlockSpec((NB, 1), lambda i: (i, 0)),
    pl.BlockSpec((NB, 1), lambda i: (i, 0)),
]

_NODE_OUT_SHAPE = [
    jax.ShapeDtypeStruct((N_PAD, DH), jnp.float32),
    jax.ShapeDtypeStruct((N_PAD, DH), jnp.float32),
    jax.ShapeDtypeStruct((N_PAD, 1), jnp.float32),
    jax.ShapeDtypeStruct((N_PAD, 1), jnp.float32),
    jax.ShapeDtypeStruct((N_PAD, 1), jnp.float32),
]


def _node_dense_body(x_ref, w_ref, as_ref, ad_ref, we_ref, aev_ref, easum_ref,
                     h0_ref, h1_ref, hs_ref, hd_ref, sh_ref):
    h = x_ref[...] @ w_ref[...]
    _node_tail(h, as_ref, ad_ref, we_ref, aev_ref, easum_ref,
               h0_ref, h1_ref, hs_ref, hd_ref, sh_ref)


def _node_dense(xp, w, asv, adv, we, aev, easum):
    grid = N_PAD // NB
    return pl.pallas_call(
        _node_dense_body,
        grid=(grid,),
        in_specs=[
            pl.BlockSpec((NB, D), lambda i: (i, 0)),
            pl.BlockSpec((D, D), lambda i: (0, 0)),
            pl.BlockSpec((D, 1), lambda i: (0, 0)),
            pl.BlockSpec((D, 1), lambda i: (0, 0)),
            pl.BlockSpec((DE, D), lambda i: (0, 0)),
            pl.BlockSpec((D, 1), lambda i: (0, 0)),
            pl.BlockSpec((1, DE), lambda i: (0, 0)),
        ],
        out_specs=_NODE_OUT_SPECS,
        out_shape=_NODE_OUT_SHAPE,
    )(xp, w, asv, adv, we, aev, easum)


def _mid_dense_body(p0_ref, p1_ref, h0_ref, h1_ref, d_ref, b_ref,
                    w_ref, as_ref, ad_ref, we_ref, aev_ref, easum_ref,
                    h0o_ref, h1o_ref, hs_ref, hd_ref, sh_ref):
    den = d_ref[...] + 1.0
    msg = jnp.concatenate(
        [p0_ref[...] + h0_ref[...], p1_ref[...] + h1_ref[...]], axis=-1)
    y = msg / den + b_ref[...]
    y = y * (1.0 / (1.0 + jnp.exp(-y)))
    h = y @ w_ref[...]
    _node_tail(h, as_ref, ad_ref, we_ref, aev_ref, easum_ref,
               h0o_ref, h1o_ref, hs_ref, hd_ref, sh_ref)


def _mid_dense(p0, p1, h0, h1, d, b2d, w, asv, adv, we, aev, easum):
    grid = N_PAD // NB
    return pl.pallas_call(
        _mid_dense_body,
        grid=(grid,),
        in_specs=[
            pl.BlockSpec((NB, DH), lambda i: (i, 0)),
            pl.BlockSpec((NB, DH), lambda i: (i, 0)),
            pl.BlockSpec((NB, DH), lambda i: (i, 0)),
            pl.BlockSpec((NB, DH), lambda i: (i, 0)),
            pl.BlockSpec((NB, 1), lambda i: (i, 0)),
            pl.BlockSpec((1, D), lambda i: (0, 0)),
            pl.BlockSpec((D, D), lambda i: (0, 0)),
            pl.BlockSpec((D, 1), lambda i: (0, 0)),
            pl.BlockSpec((D, 1), lambda i: (0, 0)),
            pl.BlockSpec((DE, D), lambda i: (0, 0)),
            pl.BlockSpec((D, 1), lambda i: (0, 0)),
            pl.BlockSpec((1, DE), lambda i: (0, 0)),
        ],
        out_specs=_NODE_OUT_SPECS,
        out_shape=_NODE_OUT_SHAPE,
    )(p0, p1, h0, h1, d, b2d, w, asv, adv, we, aev, easum)


def _sc_layer_body(hs_hbm, hd_hbm, sh_hbm, eatt_hbm, src_hbm, dst_hbm,
                   h0_hbm, h1_hbm, out_hbm, den_hbm,
                   hs_v, hd_v, sh_v, eatt_v, src_v, dst_v,
                   ex0, ex1, rows0, rows1,
                   out_sh, den_sh,
                   gsem0, gsem1, ssem0, ssem1, dsem0, dsem1):
    c = lax.axis_index("c")
    s = lax.axis_index("s")
    cbase = s * CPT
    rbufs = (rows0, rows1)
    exbufs = (ex0, ex1)
    gsems = (gsem0, gsem1)
    ssems = (ssem0, ssem1)
    dsems = (dsem0, dsem1)

    pltpu.sync_copy(hs_hbm, hs_v)
    pltpu.sync_copy(hd_hbm, hd_v)
    pltpu.sync_copy(sh_hbm, sh_v)

    def _zrow(i, _):
        for k in range(DH // L):
            rows0[i, pl.ds(k * L, L)] = jnp.zeros((L,), jnp.float32)
        return 0

    lax.fori_loop(0, K, _zrow, 0)
    rbase = s * ROWS_PER_TILE
    for t in range(ROWS_PER_TILE // K):
        pltpu.sync_copy(rows0, out_sh.at[pl.ds(rbase + t * K, K)])

    @pl.when(c == 0)
    def _():
        for t in range(ROWS_PER_TILE // DH):
            pltpu.sync_copy(rows0.at[0],
                            den_sh.at[pl.ds(rbase + t * DH, DH)])

    plsc.subcore_barrier()

    def _issue_gather(j, buf, sem):
        @pl.when(c == 0)
        def _():
            pltpu.async_copy(h0_hbm.at[src_v.at[j]], buf, sem)

        @pl.when(c == 1)
        def _():
            pltpu.async_copy(h1_hbm.at[src_v.at[j]], buf, sem)

    def _chunk(j, b):
        nb = 1 - b
        buf = rbufs[b]
        exb = exbufs[b]

        @pl.when(j >= 1)
        def _():
            pltpu.make_async_copy(rbufs[nb], out_sh.at[dst_v.at[0]],
                                  ssems[nb]).wait()

        @pl.when(j + 1 < HCPT)
        def _():
            _issue_gather(j + 1, rbufs[nb], gsems[nb])

        @pl.when(jnp.logical_and(c == 0, j >= 2))
        def _():
            pltpu.make_async_copy(exb, den_sh.at[dst_v.at[0]],
                                  dsems[b]).wait()

        for k in range(K // L):
            sl = pl.ds(k * L, L)
            isrc = src_v[j, sl]
            idst = dst_v[j, sl]
            av = plsc.load_gather(hs_v, [isrc])
            bv = plsc.load_gather(hd_v, [idst])
            shv = plsc.load_gather(sh_v, [idst])
            al = av + bv + eatt_v[j, sl]
            al = jnp.where(al >= 0.0, al, 0.2 * al)
            exb[sl] = jnp.exp(jnp.minimum(al - shv, 80.0))

        @pl.when(c == 0)
        def _():
            pltpu.async_copy(exb, den_sh.at[dst_v.at[j]], dsems[b], add=True)

        pltpu.make_async_copy(h0_hbm.at[src_v.at[0]], buf, gsems[b]).wait()

        def _scale(v, _):
            exv = exb[pl.ds(v * L, L)]
            base = v * L
            for e in range(L):
                sv = exv[e]
                for k in range(DH // L):
                    sl = pl.ds(k * L, L)
                    buf[base + e, sl] = buf[base + e, sl] * sv
            return 0

        lax.fori_loop(0, K // L, _scale, 0)
        pltpu.async_copy(buf, out_sh.at[dst_v.at[j]], ssems[b], add=True)

    def _pair(i, _):
        _chunk(i * 2, 0)
        _chunk(i * 2 + 1, 1)
        return 0

    for p in range(NPASS):
        pbase = cbase + p * HCPT
        pltpu.sync_copy(eatt_hbm.at[pl.ds(pbase, HCPT)], eatt_v)
        pltpu.sync_copy(src_hbm.at[pl.ds(pbase, HCPT)], src_v)
        pltpu.sync_copy(dst_hbm.at[pl.ds(pbase, HCPT)], dst_v)
        _issue_gather(0, rows0, gsem0)
        lax.fori_loop(0, HCPT // 2, _pair, 0)
        pltpu.make_async_copy(rows1, out_sh.at[dst_v.at[0]], ssem1).wait()

        @pl.when(c == 0)
        def _():
            pltpu.make_async_copy(ex0, den_sh.at[dst_v.at[0]], dsem0).wait()
            pltpu.make_async_copy(ex1, den_sh.at[dst_v.at[0]], dsem1).wait()

    plsc.subcore_barrier()

    pltpu.sync_copy(out_sh.at[pl.ds(rbase, ROWS_PER_TILE)],
                    out_hbm.at[c, pl.ds(rbase, ROWS_PER_TILE)])

    @pl.when(c == 0)
    def _():
        pltpu.sync_copy(den_sh.at[pl.ds(rbase, ROWS_PER_TILE)],
                        den_hbm.at[pl.ds(rbase, ROWS_PER_TILE)])


_sc_layer = pl.kernel(
    _sc_layer_body,
    out_type=[
        jax.ShapeDtypeStruct((NC, N_PAD, DH), jnp.float32),
        jax.ShapeDtypeStruct((N_PAD,), jnp.float32),
    ],
    mesh=plsc.VectorSubcoreMesh(core_axis_name="c", subcore_axis_name="s"),
    compiler_params=pltpu.CompilerParams(
        use_tc_tiling_on_sc=False, needs_layout_passes=False),
    scratch_types=[
        pltpu.VMEM((N_PAD,), jnp.float32),
        pltpu.VMEM((N_PAD,), jnp.float32),
        pltpu.VMEM((N_PAD,), jnp.float32),
        pltpu.VMEM((HCPT, K), jnp.float32),
        pltpu.VMEM((HCPT, K), jnp.int32),
        pltpu.VMEM((HCPT, K), jnp.int32),
        pltpu.VMEM((K,), jnp.float32),
        pltpu.VMEM((K,), jnp.float32),
        pltpu.VMEM((K, DH), jnp.float32),
        pltpu.VMEM((K, DH), jnp.float32),
        pltpu.VMEM_SHARED((N_PAD, DH), jnp.float32),
        pltpu.VMEM_SHARED((N_PAD,), jnp.float32),
        pltpu.SemaphoreType.DMA,
        pltpu.SemaphoreType.DMA,
        pltpu.SemaphoreType.DMA,
        pltpu.SemaphoreType.DMA,
        pltpu.SemaphoreType.DMA,
        pltpu.SemaphoreType.DMA,
    ],
)


def _epilogue_pool_body(p0_ref, p1_ref, h0_ref, h1_ref, d_ref, b_ref, bid_ref,
                        pooled_ref, cnt_ref):
    i = pl.program_id(0)

    @pl.when(i == 0)
    def _():
        pooled_ref[...] = jnp.zeros_like(pooled_ref)
        cnt_ref[...] = jnp.zeros_like(cnt_ref)

    den = d_ref[...] + 1.0
    msg = jnp.concatenate(
        [p0_ref[...] + h0_ref[...], p1_ref[...] + h1_ref[...]], axis=-1)
    y = msg / den + b_ref[...]
    y = y * (1.0 / (1.0 + jnp.exp(-y)))
    bid = bid_ref[...]
    ones = jnp.ones_like(y)
    for g in range(G):
        m = bid == g
        pooled_ref[g:g + 1, :] += jnp.sum(jnp.where(m, y, 0.0), axis=0,
                                          keepdims=True)
        cnt_ref[g:g + 1, :] += jnp.sum(jnp.where(m, ones, 0.0), axis=0,
                                       keepdims=True)

    @pl.when(i == pl.num_programs(0) - 1)
    def _():
        pooled_ref[...] = pooled_ref[...] / jnp.maximum(cnt_ref[...], 1.0)


def _epilogue_pool(p0, p1, h0, h1, d, b2d, bid2d):
    grid = N_PAD // NB
    return pl.pallas_call(
        _epilogue_pool_body,
        grid=(grid,),
        in_specs=[
            pl.BlockSpec((NB, DH), lambda i: (i, 0)),
            pl.BlockSpec((NB, DH), lambda i: (i, 0)),
            pl.BlockSpec((NB, DH), lambda i: (i, 0)),
            pl.BlockSpec((NB, DH), lambda i: (i, 0)),
            pl.BlockSpec((NB, 1), lambda i: (i, 0)),
            pl.BlockSpec((1, D), lambda i: (0, 0)),
            pl.BlockSpec((NB, 1), lambda i: (i, 0)),
        ],
        out_specs=pl.BlockSpec((G, D), lambda i: (0, 0)),
        out_shape=jax.ShapeDtypeStruct((G, D), jnp.float32),
        scratch_shapes=[pltpu.VMEM((G, D), jnp.float32)],
    )(p0, p1, h0, h1, d, b2d, bid2d)


def kernel(x, edge_index, edge_attr, batch,
           W1, as1, ad1, We1, ae1, b1, W2, as2, ad2, We2, ae2, b2):
    f32 = jnp.float32
    xp = jnp.zeros((N_PAD, D), f32).at[:N].set(x)
    pad_idx = jnp.full((E_PAD - E,), N_PAD - 1, jnp.int32)
    src2d = jnp.concatenate([edge_index[0], pad_idx]).reshape(E_PAD // K, K)
    dst2d = jnp.concatenate([edge_index[1], pad_idx]).reshape(E_PAD // K, K)
    bid2d = jnp.full((N_PAD, 1), G, jnp.int32).at[:N, 0].set(batch)

    eatt1, eatt2, easum = _edge_dense(edge_attr, We1, ae1.reshape(D, 1),
                                      We2, ae2.reshape(D, 1))
    zpad = jnp.zeros((E_PAD // K - E // K, K), f32)
    e1_2d = jnp.concatenate([eatt1.reshape(E // K, K), zpad], axis=0)
    e2_2d = jnp.concatenate([eatt2.reshape(E // K, K), zpad], axis=0)

    h0, h1, hs, hd, sh = _node_dense(xp, W1, as1.reshape(D, 1),
                                     ad1.reshape(D, 1), We1,
                                     ae1.reshape(D, 1), easum)
    parts1, den1 = _sc_layer(hs.reshape(N_PAD), hd.reshape(N_PAD),
                             sh.reshape(N_PAD), e1_2d, src2d, dst2d, h0, h1)
    g0, g1, hs2, hd2, sh2 = _mid_dense(parts1[0], parts1[1], h0, h1,
                                       den1.reshape(N_PAD, 1),
                                       b1.reshape(1, D), W2,
                                       as2.reshape(D, 1), ad2.reshape(D, 1),
                                       We2, ae2.reshape(D, 1), easum)
    parts2, den2 = _sc_layer(hs2.reshape(N_PAD), hd2.reshape(N_PAD),
                             sh2.reshape(N_PAD), e2_2d, src2d, dst2d, g0, g1)
    pooled = _epilogue_pool(parts2[0], parts2[1], g0, g1,
                            den2.reshape(N_PAD, 1), b2.reshape(1, D), bid2d)
    return pooled


# speedup vs baseline: 1.3122x; 1.2763x over previous
import jax
import jax.numpy as jnp
from jax import lax
from jax.experimental import pallas as pl
from jax.experimental.pallas import tpu as pltpu
from jax.experimental.pallas import tpu_sc as plsc

N = 10000
E = 320000
D = 128
DE = 16
G = 16

NC = 2
NS = 16
L = 16

N_PAD = 10240
E_PAD = 327680
EPT = E_PAD // NS
K = 128
CPT = EPT // K
NPASS = 2
HCPT = CPT // NPASS
DH = D // NC
ROWS_PER_TILE = N_PAD // NS

NB = 256
EB = 2000


def _edge_dense_body(ea_ref, we_ref, aev_ref, e_ref, easum_ref):
    i = pl.program_id(0)
    ea = ea_ref[...]
    e_ref[...] = ea @ (we_ref[...] @ aev_ref[...])

    @pl.when(i == 0)
    def _():
        easum_ref[...] = jnp.zeros_like(easum_ref)

    easum_ref[...] += jnp.sum(ea, axis=0, keepdims=True)


def _edge_dense(ea, we, aev):
    grid = E // EB
    return pl.pallas_call(
        _edge_dense_body,
        grid=(grid,),
        in_specs=[
            pl.BlockSpec((EB, DE), lambda i: (i, 0)),
            pl.BlockSpec((DE, D), lambda i: (0, 0)),
            pl.BlockSpec((D, 1), lambda i: (0, 0)),
        ],
        out_specs=[
            pl.BlockSpec((EB, 1), lambda i: (i, 0)),
            pl.BlockSpec((1, DE), lambda i: (0, 0)),
        ],
        out_shape=[
            jax.ShapeDtypeStruct((E, 1), jnp.float32),
            jax.ShapeDtypeStruct((1, DE), jnp.float32),
        ],
    )(ea, we, aev)


def _node_tail(h, as_ref, ad_ref, we_ref, aev_ref, easum_ref,
               h0_ref, h1_ref, hs_ref, hd_ref, sh_ref):
    h0_ref[...] = h[:, :DH]
    h1_ref[...] = h[:, DH:]
    hs = h @ as_ref[...]
    hd = h @ ad_ref[...]
    w_e = we_ref[...] @ aev_ref[...]
    c = (easum_ref[...] @ w_e)[0, 0] * (1.0 / E)
    t = hs + hd + c
    hs_ref[...] = hs
    hd_ref[...] = hd
    sh_ref[...] = jnp.where(t >= 0.0, t, 0.2 * t)


_NODE_OUT_SPECS = [
    pl.BlockSpec((NB, DH), lambda i: (i, 0)),
    pl.BlockSpec((NB, DH), lambda i: (i, 0)),
    pl.BlockSpec((NB, 1), lambda i: (i, 0)),
    pl.BlockSpec((NB, 1), lambda i: (i, 0)),
    pl.BlockSpec((NB, 1), lambda i: (i, 0)),
]

_NODE_OUT_SHAPE = [
    jax.ShapeDtypeStruct((N_PAD, DH), jnp.float32),
    jax.ShapeDtypeStruct((N_PAD, DH), jnp.float32),
    jax.ShapeDtypeStruct((N_PAD, 1), jnp.float32),
    jax.ShapeDtypeStruct((N_PAD, 1), jnp.float32),
    jax.ShapeDtypeStruct((N_PAD, 1), jnp.float32),
]


def _node_dense_body(x_ref, w_ref, as_ref, ad_ref, we_ref, aev_ref, easum_ref,
                     h0_ref, h1_ref, hs_ref, hd_ref, sh_ref):
    h = x_ref[...] @ w_ref[...]
    _node_tail(h, as_ref, ad_ref, we_ref, aev_ref, easum_ref,
               h0_ref, h1_ref, hs_ref, hd_ref, sh_ref)


def _node_dense(xp, w, asv, adv, we, aev, easum):
    grid = N_PAD // NB
    return pl.pallas_call(
        _node_dense_body,
        grid=(grid,),
        in_specs=[
            pl.BlockSpec((NB, D), lambda i: (i, 0)),
            pl.BlockSpec((D, D), lambda i: (0, 0)),
            pl.BlockSpec((D, 1), lambda i: (0, 0)),
            pl.BlockSpec((D, 1), lambda i: (0, 0)),
            pl.BlockSpec((DE, D), lambda i: (0, 0)),
            pl.BlockSpec((D, 1), lambda i: (0, 0)),
            pl.BlockSpec((1, DE), lambda i: (0, 0)),
        ],
        out_specs=_NODE_OUT_SPECS,
        out_shape=_NODE_OUT_SHAPE,
    )(xp, w, asv, adv, we, aev, easum)


def _mid_dense_body(p0_ref, p1_ref, h0_ref, h1_ref, d_ref, b_ref,
                    w_ref, as_ref, ad_ref, we_ref, aev_ref, easum_ref,
                    h0o_ref, h1o_ref, hs_ref, hd_ref, sh_ref):
    den = d_ref[...] + 1.0
    msg = jnp.concatenate(
        [p0_ref[...] + h0_ref[...], p1_ref[...] + h1_ref[...]], axis=-1)
    y = msg / den + b_ref[...]
    y = y * (1.0 / (1.0 + jnp.exp(-y)))
    h = y @ w_ref[...]
    _node_tail(h, as_ref, ad_ref, we_ref, aev_ref, easum_ref,
               h0o_ref, h1o_ref, hs_ref, hd_ref, sh_ref)


def _mid_dense(p0, p1, h0, h1, d, b2d, w, asv, adv, we, aev, easum):
    grid = N_PAD // NB
    return pl.pallas_call(
        _mid_dense_body,
        grid=(grid,),
        in_specs=[
            pl.BlockSpec((NB, DH), lambda i: (i, 0)),
            pl.BlockSpec((NB, DH), lambda i: (i, 0)),
            pl.BlockSpec((NB, DH), lambda i: (i, 0)),
            pl.BlockSpec((NB, DH), lambda i: (i, 0)),
            pl.BlockSpec((NB, 1), lambda i: (i, 0)),
            pl.BlockSpec((1, D), lambda i: (0, 0)),
            pl.BlockSpec((D, D), lambda i: (0, 0)),
            pl.BlockSpec((D, 1), lambda i: (0, 0)),
            pl.BlockSpec((D, 1), lambda i: (0, 0)),
            pl.BlockSpec((DE, D), lambda i: (0, 0)),
            pl.BlockSpec((D, 1), lambda i: (0, 0)),
            pl.BlockSpec((1, DE), lambda i: (0, 0)),
        ],
        out_specs=_NODE_OUT_SPECS,
        out_shape=_NODE_OUT_SHAPE,
    )(p0, p1, h0, h1, d, b2d, w, asv, adv, we, aev, easum)


def _sc_layer_body(hs_hbm, hd_hbm, sh_hbm, eatt_hbm, src_hbm, dst_hbm,
                   h0_hbm, h1_hbm, out_hbm, den_hbm,
                   hs_v, hd_v, sh_v, eatt_v, src_v, dst_v,
                   ex0, ex1, rows0, rows1,
                   out_sh, den_sh,
                   gsem0, gsem1, ssem0, ssem1, dsem0, dsem1):
    c = lax.axis_index("c")
    s = lax.axis_index("s")
    cbase = s * CPT
    rbufs = (rows0, rows1)
    exbufs = (ex0, ex1)
    gsems = (gsem0, gsem1)
    ssems = (ssem0, ssem1)
    dsems = (dsem0, dsem1)

    pltpu.sync_copy(hs_hbm, hs_v)
    pltpu.sync_copy(hd_hbm, hd_v)
    pltpu.sync_copy(sh_hbm, sh_v)

    def _zrow(i, _):
        for k in range(DH // L):
            rows0[i, pl.ds(k * L, L)] = jnp.zeros((L,), jnp.float32)
        return 0

    lax.fori_loop(0, K, _zrow, 0)
    rbase = s * ROWS_PER_TILE
    for t in range(ROWS_PER_TILE // K):
        pltpu.sync_copy(rows0, out_sh.at[pl.ds(rbase + t * K, K)])

    @pl.when(c == 0)
    def _():
        for t in range(ROWS_PER_TILE // DH):
            pltpu.sync_copy(rows0.at[0],
                            den_sh.at[pl.ds(rbase + t * DH, DH)])

    plsc.subcore_barrier()

    def _issue_gather(j, buf, sem):
        @pl.when(c == 0)
        def _():
            pltpu.async_copy(h0_hbm.at[src_v.at[j]], buf, sem)

        @pl.when(c == 1)
        def _():
            pltpu.async_copy(h1_hbm.at[src_v.at[j]], buf, sem)

    def _chunk(j, b):
        nb = 1 - b
        buf = rbufs[b]
        exb = exbufs[b]

        @pl.when(j >= 1)
        def _():
            pltpu.make_async_copy(rbufs[nb], out_sh.at[dst_v.at[0]],
                                  ssems[nb]).wait()

        @pl.when(j + 1 < HCPT)
        def _():
            _issue_gather(j + 1, rbufs[nb], gsems[nb])

        @pl.when(jnp.logical_and(c == 0, j >= 2))
        def _():
            pltpu.make_async_copy(exb, den_sh.at[dst_v.at[0]],
                                  dsems[b]).wait()

        for k in range(K // L):
            sl = pl.ds(k * L, L)
            isrc = src_v[j, sl]
            idst = dst_v[j, sl]
            av = plsc.load_gather(hs_v, [isrc])
            bv = plsc.load_gather(hd_v, [idst])
            shv = plsc.load_gather(sh_v, [idst])
            al = av + bv + eatt_v[j, sl]
            al = jnp.where(al >= 0.0, al, 0.2 * al)
            exb[sl] = jnp.exp(jnp.minimum(al - shv, 80.0))

        @pl.when(c == 0)
        def _():
            pltpu.async_copy(exb, den_sh.at[dst_v.at[j]], dsems[b], add=True)

        pltpu.make_async_copy(h0_hbm.at[src_v.at[0]], buf, gsems[b]).wait()

        for v in range(K // L):
            exv = exb[pl.ds(v * L, L)]
            base = v * L
            for e in range(L):
                sv = exv[e]
                for k in range(DH // L):
                    sl = pl.ds(k * L, L)
                    buf[base + e, sl] = buf[base + e, sl] * sv
        pltpu.async_copy(buf, out_sh.at[dst_v.at[j]], ssems[b], add=True)

    def _pair(i, _):
        _chunk(i * 2, 0)
        _chunk(i * 2 + 1, 1)
        return 0

    for p in range(NPASS):
        pbase = cbase + p * HCPT
        pltpu.sync_copy(eatt_hbm.at[pl.ds(pbase, HCPT)], eatt_v)
        pltpu.sync_copy(src_hbm.at[pl.ds(pbase, HCPT)], src_v)
        pltpu.sync_copy(dst_hbm.at[pl.ds(pbase, HCPT)], dst_v)
        _issue_gather(0, rows0, gsem0)
        lax.fori_loop(0, HCPT // 2, _pair, 0)
        pltpu.make_async_copy(rows1, out_sh.at[dst_v.at[0]], ssem1).wait()

        @pl.when(c == 0)
        def _():
            pltpu.make_async_copy(ex0, den_sh.at[dst_v.at[0]], dsem0).wait()
            pltpu.make_async_copy(ex1, den_sh.at[dst_v.at[0]], dsem1).wait()

    plsc.subcore_barrier()

    pltpu.sync_copy(out_sh.at[pl.ds(rbase, ROWS_PER_TILE)],
                    out_hbm.at[c, pl.ds(rbase, ROWS_PER_TILE)])

    @pl.when(c == 0)
    def _():
        pltpu.sync_copy(den_sh.at[pl.ds(rbase, ROWS_PER_TILE)],
                        den_hbm.at[pl.ds(rbase, ROWS_PER_TILE)])


_sc_layer = pl.kernel(
    _sc_layer_body,
    out_type=[
        jax.ShapeDtypeStruct((NC, N_PAD, DH), jnp.float32),
        jax.ShapeDtypeStruct((N_PAD,), jnp.float32),
    ],
    mesh=plsc.VectorSubcoreMesh(core_axis_name="c", subcore_axis_name="s"),
    compiler_params=pltpu.CompilerParams(
        use_tc_tiling_on_sc=False, needs_layout_passes=False),
    scratch_types=[
        pltpu.VMEM((N_PAD,), jnp.float32),
        pltpu.VMEM((N_PAD,), jnp.float32),
        pltpu.VMEM((N_PAD,), jnp.float32),
        pltpu.VMEM((HCPT, K), jnp.float32),
        pltpu.VMEM((HCPT, K), jnp.int32),
        pltpu.VMEM((HCPT, K), jnp.int32),
        pltpu.VMEM((K,), jnp.float32),
        pltpu.VMEM((K,), jnp.float32),
        pltpu.VMEM((K, DH), jnp.float32),
        pltpu.VMEM((K, DH), jnp.float32),
        pltpu.VMEM_SHARED((N_PAD, DH), jnp.float32),
        pltpu.VMEM_SHARED((N_PAD,), jnp.float32),
        pltpu.SemaphoreType.DMA,
        pltpu.SemaphoreType.DMA,
        pltpu.SemaphoreType.DMA,
        pltpu.SemaphoreType.DMA,
        pltpu.SemaphoreType.DMA,
        pltpu.SemaphoreType.DMA,
    ],
)


def _epilogue_pool_body(p0_ref, p1_ref, h0_ref, h1_ref, d_ref, b_ref, bid_ref,
                        pooled_ref, cnt_ref):
    i = pl.program_id(0)

    @pl.when(i == 0)
    def _():
        pooled_ref[...] = jnp.zeros_like(pooled_ref)
        cnt_ref[...] = jnp.zeros_like(cnt_ref)

    den = d_ref[...] + 1.0
    msg = jnp.concatenate(
        [p0_ref[...] + h0_ref[...], p1_ref[...] + h1_ref[...]], axis=-1)
    y = msg / den + b_ref[...]
    y = y * (1.0 / (1.0 + jnp.exp(-y)))
    bid = bid_ref[...]
    ones = jnp.ones_like(y)
    for g in range(G):
        m = bid == g
        pooled_ref[g:g + 1, :] += jnp.sum(jnp.where(m, y, 0.0), axis=0,
                                          keepdims=True)
        cnt_ref[g:g + 1, :] += jnp.sum(jnp.where(m, ones, 0.0), axis=0,
                                       keepdims=True)

    @pl.when(i == pl.num_programs(0) - 1)
    def _():
        pooled_ref[...] = pooled_ref[...] / jnp.maximum(cnt_ref[...], 1.0)


def _epilogue_pool(p0, p1, h0, h1, d, b2d, bid2d):
    grid = N_PAD // NB
    return pl.pallas_call(
        _epilogue_pool_body,
        grid=(grid,),
        in_specs=[
            pl.BlockSpec((NB, DH), lambda i: (i, 0)),
            pl.BlockSpec((NB, DH), lambda i: (i, 0)),
            pl.BlockSpec((NB, DH), lambda i: (i, 0)),
            pl.BlockSpec((NB, DH), lambda i: (i, 0)),
            pl.BlockSpec((NB, 1), lambda i: (i, 0)),
            pl.BlockSpec((1, D), lambda i: (0, 0)),
            pl.BlockSpec((NB, 1), lambda i: (i, 0)),
        ],
        out_specs=pl.BlockSpec((G, D), lambda i: (0, 0)),
        out_shape=jax.ShapeDtypeStruct((G, D), jnp.float32),
        scratch_shapes=[pltpu.VMEM((G, D), jnp.float32)],
    )(p0, p1, h0, h1, d, b2d, bid2d)


def kernel(x, edge_index, edge_attr, batch,
           W1, as1, ad1, We1, ae1, b1, W2, as2, ad2, We2, ae2, b2):
    f32 = jnp.float32
    xp = jnp.zeros((N_PAD, D), f32).at[:N].set(x)
    pad_idx = jnp.full((E_PAD - E,), N_PAD - 1, jnp.int32)
    src2d = jnp.concatenate([edge_index[0], pad_idx]).reshape(E_PAD // K, K)
    dst2d = jnp.concatenate([edge_index[1], pad_idx]).reshape(E_PAD // K, K)
    bid2d = jnp.full((N_PAD, 1), G, jnp.int32).at[:N, 0].set(batch)

    eatt1, easum = _edge_dense(edge_attr, We1, ae1.reshape(D, 1))
    zpad = jnp.zeros((E_PAD // K - E // K, K), f32)
    e1_2d = jnp.concatenate([eatt1.reshape(E // K, K), zpad], axis=0)

    h0, h1, hs, hd, sh = _node_dense(xp, W1, as1.reshape(D, 1),
                                     ad1.reshape(D, 1), We1,
                                     ae1.reshape(D, 1), easum)
    parts1, den1 = _sc_layer(hs.reshape(N_PAD), hd.reshape(N_PAD),
                             sh.reshape(N_PAD), e1_2d, src2d, dst2d, h0, h1)
    eatt2, _ = _edge_dense(edge_attr, We2, ae2.reshape(D, 1))
    e2_2d = jnp.concatenate([eatt2.reshape(E // K, K), zpad], axis=0)
    g0, g1, hs2, hd2, sh2 = _mid_dense(parts1[0], parts1[1], h0, h1,
                                       den1.reshape(N_PAD, 1),
                                       b1.reshape(1, D), W2,
                                       as2.reshape(D, 1), ad2.reshape(D, 1),
                                       We2, ae2.reshape(D, 1), easum)
    parts2, den2 = _sc_layer(hs2.reshape(N_PAD), hd2.reshape(N_PAD),
                             sh2.reshape(N_PAD), e2_2d, src2d, dst2d, g0, g1)
    pooled = _epilogue_pool(parts2[0], parts2[1], g0, g1,
                            den2.reshape(N_PAD, 1), b2.reshape(1, D), bid2d)
    return pooled


# speedup vs baseline: 1.5478x; 1.1796x over previous
import jax
import jax.numpy as jnp
from jax import lax
from jax.experimental import pallas as pl
from jax.experimental.pallas import tpu as pltpu
from jax.experimental.pallas import tpu_sc as plsc

N = 10000
E = 320000
D = 128
DE = 16
G = 16

NC = 2
NS = 16
L = 16

N_PAD = 10240
E_PAD = 327680
EPT = E_PAD // NS
K = 128
CPT = EPT // K
NPASS = 2
HCPT = CPT // NPASS
DH = D // NC
ROWS_PER_TILE = N_PAD // NS

NB = 256
EB = 2000


E8 = E // 8
E8_PAD = E_PAD // 8
EBR = 512


def _eatt_dense_body(ea2_ref, wet_ref, aet_ref, rep_ref, csum_ref):
    i = pl.program_id(0)

    @pl.when(i == 0)
    def _():
        csum_ref[...] = jnp.zeros_like(csum_ref)

    w_t = aet_ref[...] @ wet_ref[...]
    wtile = jnp.concatenate([w_t] * 8, axis=1)
    prod = ea2_ref[...] * wtile
    ii = lax.broadcasted_iota(jnp.int32, (D, D), 0) // DE
    jj = lax.broadcasted_iota(jnp.int32, (D, D), 1) // DE
    bo = jnp.where(ii == jj, 1.0, 0.0)
    rep_ref[...] = prod @ bo
    csum_ref[...] += jnp.sum(prod, axis=0, keepdims=True)


def _eatt_dense(ea2p, wet, aet):
    grid = E8_PAD // EBR
    return pl.pallas_call(
        _eatt_dense_body,
        grid=(grid,),
        in_specs=[
            pl.BlockSpec((EBR, D), lambda i: (i, 0)),
            pl.BlockSpec((D, DE), lambda i: (0, 0)),
            pl.BlockSpec((1, D), lambda i: (0, 0)),
        ],
        out_specs=[
            pl.BlockSpec((EBR, D), lambda i: (i, 0)),
            pl.BlockSpec((1, D), lambda i: (0, 0)),
        ],
        out_shape=[
            jax.ShapeDtypeStruct((E8_PAD, D), jnp.float32),
            jax.ShapeDtypeStruct((1, D), jnp.float32),
        ],
    )(ea2p, wet, aet)


def _node_tail(h, as_ref, ad_ref, h0_ref, h1_ref, hs_ref, hd_ref, sh_ref):
    h0_ref[...] = h[:, :DH]
    h1_ref[...] = h[:, DH:]
    hs = h @ as_ref[...]
    hd = h @ ad_ref[...]
    t = hs + hd
    hs_ref[...] = hs
    hd_ref[...] = hd
    sh_ref[...] = jnp.where(t >= 0.0, t, 0.2 * t)


_NODE_OUT_SPECS = [
    pl.BlockSpec((NB, DH), lambda i: (i, 0)),
    pl.BlockSpec((NB, DH), lambda i: (i, 0)),
    pl.BlockSpec((NB, 1), lambda i: (i, 0)),
    pl.BlockSpec((NB, 1), lambda i: (i, 0)),
    pl.BlockSpec((NB, 1), lambda i: (i, 0)),
]

_NODE_OUT_SHAPE = [
    jax.ShapeDtypeStruct((N_PAD, DH), jnp.float32),
    jax.ShapeDtypeStruct((N_PAD, DH), jnp.float32),
    jax.ShapeDtypeStruct((N_PAD, 1), jnp.float32),
    jax.ShapeDtypeStruct((N_PAD, 1), jnp.float32),
    jax.ShapeDtypeStruct((N_PAD, 1), jnp.float32),
]


def _node_dense_body(x_ref, w_ref, as_ref, ad_ref,
                     h0_ref, h1_ref, hs_ref, hd_ref, sh_ref):
    h = x_ref[...] @ w_ref[...]
    _node_tail(h, as_ref, ad_ref, h0_ref, h1_ref, hs_ref, hd_ref, sh_ref)


def _node_dense(xp, w, asv, adv):
    grid = N_PAD // NB
    return pl.pallas_call(
        _node_dense_body,
        grid=(grid,),
        in_specs=[
            pl.BlockSpec((NB, D), lambda i: (i, 0)),
            pl.BlockSpec((D, D), lambda i: (0, 0)),
            pl.BlockSpec((D, 1), lambda i: (0, 0)),
            pl.BlockSpec((D, 1), lambda i: (0, 0)),
        ],
        out_specs=_NODE_OUT_SPECS,
        out_shape=_NODE_OUT_SHAPE,
    )(xp, w, asv, adv)


def _layer_out(p0_ref, p1_ref, h0_ref, h1_ref, hs_ref, hd_ref, sh_ref,
               csum_ref, d_ref, b_ref):
    c = jnp.sum(csum_ref[...]) * (1.0 / E)
    t = hs_ref[...] + hd_ref[...] + c
    t = jnp.where(t >= 0.0, t, 0.2 * t)
    selfex = jnp.exp(jnp.minimum(t - sh_ref[...], 80.0))
    den = d_ref[...] + selfex
    msg = jnp.concatenate(
        [p0_ref[...] + h0_ref[...] * selfex,
         p1_ref[...] + h1_ref[...] * selfex], axis=-1)
    y = msg / den + b_ref[...]
    return y * (1.0 / (1.0 + jnp.exp(-y)))


_LAYER_IN_SPECS = [
    pl.BlockSpec((NB, DH), lambda i: (i, 0)),
    pl.BlockSpec((NB, DH), lambda i: (i, 0)),
    pl.BlockSpec((NB, DH), lambda i: (i, 0)),
    pl.BlockSpec((NB, DH), lambda i: (i, 0)),
    pl.BlockSpec((NB, 1), lambda i: (i, 0)),
    pl.BlockSpec((NB, 1), lambda i: (i, 0)),
    pl.BlockSpec((NB, 1), lambda i: (i, 0)),
    pl.BlockSpec((1, D), lambda i: (0, 0)),
    pl.BlockSpec((NB, 1), lambda i: (i, 0)),
    pl.BlockSpec((1, D), lambda i: (0, 0)),
]


def _mid_dense_body(p0_ref, p1_ref, h0_ref, h1_ref, hs_ref, hd_ref, sh_ref,
                    csum_ref, d_ref, b_ref, w_ref, as_ref, ad_ref,
                    h0o_ref, h1o_ref, hso_ref, hdo_ref, sho_ref):
    y = _layer_out(p0_ref, p1_ref, h0_ref, h1_ref, hs_ref, hd_ref, sh_ref,
                   csum_ref, d_ref, b_ref)
    h = y @ w_ref[...]
    _node_tail(h, as_ref, ad_ref, h0o_ref, h1o_ref, hso_ref, hdo_ref, sho_ref)


def _mid_dense(p0, p1, h0, h1, hs, hd, sh, csum, d, b2d, w, asv, adv):
    grid = N_PAD // NB
    return pl.pallas_call(
        _mid_dense_body,
        grid=(grid,),
        in_specs=_LAYER_IN_SPECS + [
            pl.BlockSpec((D, D), lambda i: (0, 0)),
            pl.BlockSpec((D, 1), lambda i: (0, 0)),
            pl.BlockSpec((D, 1), lambda i: (0, 0)),
        ],
        out_specs=_NODE_OUT_SPECS,
        out_shape=_NODE_OUT_SHAPE,
    )(p0, p1, h0, h1, hs, hd, sh, csum, d, b2d, w, asv, adv)


def _sc_layer_body(hs_hbm, hd_hbm, sh_hbm, rep_hbm, src_hbm, dst_hbm,
                   h0_hbm, h1_hbm, out_hbm, den_hbm,
                   hs_v, hd_v, sh_v, eb0, eb1, src_v, dst_v,
                   ex0, ex1, rows0, rows1,
                   out_sh, den_sh,
                   gsem0, gsem1, ssem0, ssem1, dsem0, dsem1, esem0, esem1):
    c = lax.axis_index("c")
    s = lax.axis_index("s")
    cbase = s * CPT
    rbufs = (rows0, rows1)
    ebufs = (eb0, eb1)
    exbufs = (ex0, ex1)
    gsems = (gsem0, gsem1)
    ssems = (ssem0, ssem1)
    dsems = (dsem0, dsem1)
    esems = (esem0, esem1)
    pat = lax.iota(jnp.int32, L) * L

    pltpu.sync_copy(hs_hbm, hs_v)
    pltpu.sync_copy(hd_hbm, hd_v)
    pltpu.sync_copy(sh_hbm, sh_v)

    def _zrow(i, _):
        for k in range(DH // L):
            rows0[i, pl.ds(k * L, L)] = jnp.zeros((L,), jnp.float32)
        return 0

    lax.fori_loop(0, K, _zrow, 0)
    rbase = s * ROWS_PER_TILE
    for t in range(ROWS_PER_TILE // K):
        pltpu.sync_copy(rows0, out_sh.at[pl.ds(rbase + t * K, K)])

    @pl.when(c == 0)
    def _():
        for t in range(ROWS_PER_TILE // DH):
            pltpu.sync_copy(rows0.at[0],
                            den_sh.at[pl.ds(rbase + t * DH, DH)])

    plsc.subcore_barrier()

    def _issue_gather(j, buf, sem):
        @pl.when(c == 0)
        def _():
            pltpu.async_copy(h0_hbm.at[src_v.at[j]], buf, sem)

        @pl.when(c == 1)
        def _():
            pltpu.async_copy(h1_hbm.at[src_v.at[j]], buf, sem)

    def _issue_eatt(grow, buf, sem):
        pltpu.async_copy(rep_hbm.at[pl.ds(grow * (K * L), K * L)], buf, sem)

    def _chunk(p, j, b):
        nb = 1 - b
        buf = rbufs[b]
        ebuf = ebufs[b]
        exb = exbufs[b]
        grow = cbase + p * HCPT + j

        @pl.when(j >= 1)
        def _():
            pltpu.make_async_copy(rbufs[nb], out_sh.at[dst_v.at[0]],
                                  ssems[nb]).wait()

        @pl.when(j + 1 < HCPT)
        def _():
            _issue_gather(j + 1, rbufs[nb], gsems[nb])
            _issue_eatt(grow + 1, ebufs[nb], esems[nb])

        @pl.when(jnp.logical_and(c == 0, j >= 2))
        def _():
            pltpu.make_async_copy(exb, den_sh.at[dst_v.at[0]],
                                  dsems[b]).wait()

        pltpu.make_async_copy(rep_hbm.at[pl.ds(0, K * L)], ebuf,
                              esems[b]).wait()
        for k in range(K // L):
            sl = pl.ds(k * L, L)
            isrc = src_v[j, sl]
            idst = dst_v[j, sl]
            av = plsc.load_gather(hs_v, [isrc])
            bv = plsc.load_gather(hd_v, [idst])
            shv = plsc.load_gather(sh_v, [idst])
            eattv = plsc.load_gather(ebuf, [pat + (2 * K * k)])
            al = av + bv + eattv
            al = jnp.where(al >= 0.0, al, 0.2 * al)
            exb[sl] = jnp.exp(jnp.minimum(al - shv, 80.0))

        @pl.when(c == 0)
        def _():
            pltpu.async_copy(exb, den_sh.at[dst_v.at[j]], dsems[b], add=True)

        pltpu.make_async_copy(h0_hbm.at[src_v.at[0]], buf, gsems[b]).wait()

        for v in range(K // L):
            exv = exb[pl.ds(v * L, L)]
            base = v * L
            for e in range(L):
                sv = exv[e]
                for k in range(DH // L):
                    sl = pl.ds(k * L, L)
                    buf[base + e, sl] = buf[base + e, sl] * sv
        pltpu.async_copy(buf, out_sh.at[dst_v.at[j]], ssems[b], add=True)

    for p in range(NPASS):
        pbase = cbase + p * HCPT
        pltpu.sync_copy(src_hbm.at[pl.ds(pbase, HCPT)], src_v)
        pltpu.sync_copy(dst_hbm.at[pl.ds(pbase, HCPT)], dst_v)
        _issue_gather(0, rows0, gsem0)
        _issue_eatt(pbase, eb0, esem0)

        def _pair(i, _, p=p):
            _chunk(p, i * 2, 0)
            _chunk(p, i * 2 + 1, 1)
            return 0

        lax.fori_loop(0, HCPT // 2, _pair, 0)
        pltpu.make_async_copy(rows1, out_sh.at[dst_v.at[0]], ssem1).wait()

        @pl.when(c == 0)
        def _():
            pltpu.make_async_copy(ex0, den_sh.at[dst_v.at[0]], dsem0).wait()
            pltpu.make_async_copy(ex1, den_sh.at[dst_v.at[0]], dsem1).wait()

    plsc.subcore_barrier()

    pltpu.sync_copy(out_sh.at[pl.ds(rbase, ROWS_PER_TILE)],
                    out_hbm.at[c, pl.ds(rbase, ROWS_PER_TILE)])

    @pl.when(c == 0)
    def _():
        pltpu.sync_copy(den_sh.at[pl.ds(rbase, ROWS_PER_TILE)],
                        den_hbm.at[pl.ds(rbase, ROWS_PER_TILE)])


_sc_layer = pl.kernel(
    _sc_layer_body,
    out_type=[
        jax.ShapeDtypeStruct((NC, N_PAD, DH), jnp.float32),
        jax.ShapeDtypeStruct((N_PAD,), jnp.float32),
    ],
    mesh=plsc.VectorSubcoreMesh(core_axis_name="c", subcore_axis_name="s"),
    compiler_params=pltpu.CompilerParams(
        use_tc_tiling_on_sc=False, needs_layout_passes=False),
    scratch_types=[
        pltpu.VMEM((N_PAD,), jnp.float32),
        pltpu.VMEM((N_PAD,), jnp.float32),
        pltpu.VMEM((N_PAD,), jnp.float32),
        pltpu.VMEM((K * L,), jnp.float32),
        pltpu.VMEM((K * L,), jnp.float32),
        pltpu.VMEM((HCPT, K), jnp.int32),
        pltpu.VMEM((HCPT, K), jnp.int32),
        pltpu.VMEM((K,), jnp.float32),
        pltpu.VMEM((K,), jnp.float32),
        pltpu.VMEM((K, DH), jnp.float32),
        pltpu.VMEM((K, DH), jnp.float32),
        pltpu.VMEM_SHARED((N_PAD, DH), jnp.float32),
        pltpu.VMEM_SHARED((N_PAD,), jnp.float32),
        pltpu.SemaphoreType.DMA,
        pltpu.SemaphoreType.DMA,
        pltpu.SemaphoreType.DMA,
        pltpu.SemaphoreType.DMA,
        pltpu.SemaphoreType.DMA,
        pltpu.SemaphoreType.DMA,
        pltpu.SemaphoreType.DMA,
        pltpu.SemaphoreType.DMA,
    ],
)


def _epilogue_pool_body(p0_ref, p1_ref, h0_ref, h1_ref, hs_ref, hd_ref,
                        sh_ref, csum_ref, d_ref, b_ref, bid_ref,
                        pooled_ref, cnt_ref):
    i = pl.program_id(0)

    @pl.when(i == 0)
    def _():
        pooled_ref[...] = jnp.zeros_like(pooled_ref)
        cnt_ref[...] = jnp.zeros_like(cnt_ref)

    y = _layer_out(p0_ref, p1_ref, h0_ref, h1_ref, hs_ref, hd_ref, sh_ref,
                   csum_ref, d_ref, b_ref)
    bid = bid_ref[...]
    ones = jnp.ones_like(y)
    for g in range(G):
        m = bid == g
        pooled_ref[g:g + 1, :] += jnp.sum(jnp.where(m, y, 0.0), axis=0,
                                          keepdims=True)
        cnt_ref[g:g + 1, :] += jnp.sum(jnp.where(m, ones, 0.0), axis=0,
                                       keepdims=True)

    @pl.when(i == pl.num_programs(0) - 1)
    def _():
        pooled_ref[...] = pooled_ref[...] / jnp.maximum(cnt_ref[...], 1.0)


def _epilogue_pool(p0, p1, h0, h1, hs, hd, sh, csum, d, b2d, bid2d):
    grid = N_PAD // NB
    return pl.pallas_call(
        _epilogue_pool_body,
        grid=(grid,),
        in_specs=_LAYER_IN_SPECS + [
            pl.BlockSpec((NB, 1), lambda i: (i, 0)),
        ],
        out_specs=pl.BlockSpec((G, D), lambda i: (0, 0)),
        out_shape=jax.ShapeDtypeStruct((G, D), jnp.float32),
        scratch_shapes=[pltpu.VMEM((G, D), jnp.float32)],
    )(p0, p1, h0, h1, hs, hd, sh, csum, d, b2d, bid2d)


def kernel(x, edge_index, edge_attr, batch,
           W1, as1, ad1, We1, ae1, b1, W2, as2, ad2, We2, ae2, b2):
    f32 = jnp.float32
    xp = jnp.zeros((N_PAD, D), f32).at[:N].set(x)
    pad_idx = jnp.full((E_PAD - E,), N_PAD - 1, jnp.int32)
    src2d = jnp.concatenate([edge_index[0], pad_idx]).reshape(E_PAD // K, K)
    dst2d = jnp.concatenate([edge_index[1], pad_idx]).reshape(E_PAD // K, K)
    bid2d = jnp.full((N_PAD, 1), G, jnp.int32).at[:N, 0].set(batch)

    ea2p = jnp.zeros((E8_PAD, D), f32).at[:E8].set(edge_attr.reshape(E8, D))

    rep1, csum1 = _eatt_dense(ea2p, We1.T, ae1.reshape(1, D))
    h0, h1, hs, hd, sh = _node_dense(xp, W1, as1.reshape(D, 1),
                                     ad1.reshape(D, 1))
    parts1, den1 = _sc_layer(hs.reshape(N_PAD), hd.reshape(N_PAD),
                             sh.reshape(N_PAD), rep1.reshape(E8_PAD * D),
                             src2d, dst2d, h0, h1)
    rep2, csum2 = _eatt_dense(ea2p, We2.T, ae2.reshape(1, D))
    g0, g1, hs2, hd2, sh2 = _mid_dense(parts1[0], parts1[1], h0, h1,
                                       hs, hd, sh, csum1,
                                       den1.reshape(N_PAD, 1),
                                       b1.reshape(1, D), W2,
                                       as2.reshape(D, 1), ad2.reshape(D, 1))
    parts2, den2 = _sc_layer(hs2.reshape(N_PAD), hd2.reshape(N_PAD),
                             sh2.reshape(N_PAD), rep2.reshape(E8_PAD * D),
                             src2d, dst2d, g0, g1)
    pooled = _epilogue_pool(parts2[0], parts2[1], g0, g1,
                            hs2, hd2, sh2, csum2,
                            den2.reshape(N_PAD, 1), b2.reshape(1, D), bid2d)
    return pooled


# speedup vs baseline: 1.6019x; 1.0350x over previous
import jax
import jax.numpy as jnp
from jax import lax
from jax.experimental import pallas as pl
from jax.experimental.pallas import tpu as pltpu
from jax.experimental.pallas import tpu_sc as plsc

N = 10000
E = 320000
D = 128
DE = 16
G = 16

NC = 2
NS = 16
L = 16

N_PAD = 10240
E_PAD = 327680
EPT = E_PAD // NS
K = 128
CPT = EPT // K
NPASS = 2
HCPT = CPT // NPASS
DH = D // NC
ROWS_PER_TILE = N_PAD // NS

NB = 256
EB = 2000


E8 = E // 8
EBR = 1000


def _eatt_dense_body(ea2_ref, wet_ref, aet_ref, rep_ref, csum_ref):
    i = pl.program_id(0)

    @pl.when(i == 0)
    def _():
        csum_ref[...] = jnp.zeros_like(csum_ref)

    w_t = aet_ref[...] @ wet_ref[...]
    wtile = jnp.concatenate([w_t] * 8, axis=1)
    prod = ea2_ref[...] * wtile
    ii = lax.broadcasted_iota(jnp.int32, (D, D), 0) // DE
    jj = lax.broadcasted_iota(jnp.int32, (D, D), 1) // DE
    bo = jnp.where(ii == jj, 1.0, 0.0)
    rep_ref[...] = prod @ bo
    csum_ref[...] += jnp.sum(prod, axis=0, keepdims=True)


def _eatt_dense(ea2, wet, aet):
    grid = E8 // EBR
    return pl.pallas_call(
        _eatt_dense_body,
        grid=(grid,),
        in_specs=[
            pl.BlockSpec((EBR, D), lambda i: (i, 0)),
            pl.BlockSpec((D, DE), lambda i: (0, 0)),
            pl.BlockSpec((1, D), lambda i: (0, 0)),
        ],
        out_specs=[
            pl.BlockSpec((EBR, D), lambda i: (i, 0)),
            pl.BlockSpec((1, D), lambda i: (0, 0)),
        ],
        out_shape=[
            jax.ShapeDtypeStruct((E8, D), jnp.float32),
            jax.ShapeDtypeStruct((1, D), jnp.float32),
        ],
    )(ea2, wet, aet)


def _node_tail(h, as_ref, ad_ref, h0_ref, h1_ref, hs_ref, hd_ref, sh_ref):
    h0_ref[...] = h[:, :DH]
    h1_ref[...] = h[:, DH:]
    hs = h @ as_ref[...]
    hd = h @ ad_ref[...]
    t = hs + hd
    hs_ref[...] = hs
    hd_ref[...] = hd
    sh_ref[...] = jnp.where(t >= 0.0, t, 0.2 * t)


_NODE_OUT_SPECS = [
    pl.BlockSpec((NB, DH), lambda i: (i, 0)),
    pl.BlockSpec((NB, DH), lambda i: (i, 0)),
    pl.BlockSpec((NB, 1), lambda i: (i, 0)),
    pl.BlockSpec((NB, 1), lambda i: (i, 0)),
    pl.BlockSpec((NB, 1), lambda i: (i, 0)),
]

_NODE_OUT_SHAPE = [
    jax.ShapeDtypeStruct((N_PAD, DH), jnp.float32),
    jax.ShapeDtypeStruct((N_PAD, DH), jnp.float32),
    jax.ShapeDtypeStruct((N_PAD, 1), jnp.float32),
    jax.ShapeDtypeStruct((N_PAD, 1), jnp.float32),
    jax.ShapeDtypeStruct((N_PAD, 1), jnp.float32),
]


def _node_dense_body(x_ref, w_ref, as_ref, ad_ref,
                     h0_ref, h1_ref, hs_ref, hd_ref, sh_ref):
    h = x_ref[...] @ w_ref[...]
    _node_tail(h, as_ref, ad_ref, h0_ref, h1_ref, hs_ref, hd_ref, sh_ref)


def _node_dense(xp, w, asv, adv):
    grid = N_PAD // NB
    return pl.pallas_call(
        _node_dense_body,
        grid=(grid,),
        in_specs=[
            pl.BlockSpec((NB, D), lambda i: (i, 0)),
            pl.BlockSpec((D, D), lambda i: (0, 0)),
            pl.BlockSpec((D, 1), lambda i: (0, 0)),
            pl.BlockSpec((D, 1), lambda i: (0, 0)),
        ],
        out_specs=_NODE_OUT_SPECS,
        out_shape=_NODE_OUT_SHAPE,
    )(xp, w, asv, adv)


def _layer_out(p0_ref, p1_ref, h0_ref, h1_ref, hs_ref, hd_ref, sh_ref,
               csum_ref, d_ref, b_ref):
    c = jnp.sum(csum_ref[...]) * (1.0 / E)
    t = hs_ref[...] + hd_ref[...] + c
    t = jnp.where(t >= 0.0, t, 0.2 * t)
    selfex = jnp.exp(jnp.minimum(t - sh_ref[...], 80.0))
    den = d_ref[...] + selfex
    msg = jnp.concatenate(
        [p0_ref[...] + h0_ref[...] * selfex,
         p1_ref[...] + h1_ref[...] * selfex], axis=-1)
    y = msg / den + b_ref[...]
    return y * (1.0 / (1.0 + jnp.exp(-y)))


_LAYER_IN_SPECS = [
    pl.BlockSpec((NB, DH), lambda i: (i, 0)),
    pl.BlockSpec((NB, DH), lambda i: (i, 0)),
    pl.BlockSpec((NB, DH), lambda i: (i, 0)),
    pl.BlockSpec((NB, DH), lambda i: (i, 0)),
    pl.BlockSpec((NB, 1), lambda i: (i, 0)),
    pl.BlockSpec((NB, 1), lambda i: (i, 0)),
    pl.BlockSpec((NB, 1), lambda i: (i, 0)),
    pl.BlockSpec((1, D), lambda i: (0, 0)),
    pl.BlockSpec((NB, 1), lambda i: (i, 0)),
    pl.BlockSpec((1, D), lambda i: (0, 0)),
]


def _mid_dense_body(p0_ref, p1_ref, h0_ref, h1_ref, hs_ref, hd_ref, sh_ref,
                    csum_ref, d_ref, b_ref, w_ref, as_ref, ad_ref,
                    h0o_ref, h1o_ref, hso_ref, hdo_ref, sho_ref):
    y = _layer_out(p0_ref, p1_ref, h0_ref, h1_ref, hs_ref, hd_ref, sh_ref,
                   csum_ref, d_ref, b_ref)
    h = y @ w_ref[...]
    _node_tail(h, as_ref, ad_ref, h0o_ref, h1o_ref, hso_ref, hdo_ref, sho_ref)


def _mid_dense(p0, p1, h0, h1, hs, hd, sh, csum, d, b2d, w, asv, adv):
    grid = N_PAD // NB
    return pl.pallas_call(
        _mid_dense_body,
        grid=(grid,),
        in_specs=_LAYER_IN_SPECS + [
            pl.BlockSpec((D, D), lambda i: (0, 0)),
            pl.BlockSpec((D, 1), lambda i: (0, 0)),
            pl.BlockSpec((D, 1), lambda i: (0, 0)),
        ],
        out_specs=_NODE_OUT_SPECS,
        out_shape=_NODE_OUT_SHAPE,
    )(p0, p1, h0, h1, hs, hd, sh, csum, d, b2d, w, asv, adv)


def _sc_layer_body(hs_hbm, hd_hbm, sh_hbm, rep_hbm, src_hbm, dst_hbm,
                   h0_hbm, h1_hbm, out_hbm, den_hbm,
                   hs_v, hd_v, sh_v, eb0, eb1, src_v, dst_v,
                   ex0, ex1, rows0, rows1,
                   out_sh, den_sh,
                   gsem0, gsem1, ssem0, ssem1, dsem0, dsem1, esem0, esem1):
    c = lax.axis_index("c")
    s = lax.axis_index("s")
    cbase = s * CPT
    rbufs = (rows0, rows1)
    ebufs = (eb0, eb1)
    exbufs = (ex0, ex1)
    gsems = (gsem0, gsem1)
    ssems = (ssem0, ssem1)
    dsems = (dsem0, dsem1)
    esems = (esem0, esem1)
    zi = lax.iota(jnp.int32, L)
    rpat = lax.shift_right_logical(zi, 3)
    cpat = (zi - rpat * 8) * L

    pltpu.sync_copy(hs_hbm, hs_v)
    pltpu.sync_copy(hd_hbm, hd_v)
    pltpu.sync_copy(sh_hbm, sh_v)

    def _zrow(i, _):
        for k in range(DH // L):
            rows0[i, pl.ds(k * L, L)] = jnp.zeros((L,), jnp.float32)
        return 0

    lax.fori_loop(0, K, _zrow, 0)
    rbase = s * ROWS_PER_TILE
    for t in range(ROWS_PER_TILE // K):
        pltpu.sync_copy(rows0, out_sh.at[pl.ds(rbase + t * K, K)])

    @pl.when(c == 0)
    def _():
        for t in range(ROWS_PER_TILE // DH):
            pltpu.sync_copy(rows0.at[0],
                            den_sh.at[pl.ds(rbase + t * DH, DH)])

    plsc.subcore_barrier()

    def _issue_gather(j, buf, sem):
        @pl.when(c == 0)
        def _():
            pltpu.async_copy(h0_hbm.at[src_v.at[j]], buf, sem)

        @pl.when(c == 1)
        def _():
            pltpu.async_copy(h1_hbm.at[src_v.at[j]], buf, sem)

    def _issue_eatt(grow, buf, sem):
        growc = jnp.minimum(grow, E // K - 1)
        pltpu.async_copy(rep_hbm.at[pl.ds(growc * L, L)], buf, sem)

    def _chunk(p, j, b):
        nb = 1 - b
        buf = rbufs[b]
        ebuf = ebufs[b]
        exb = exbufs[b]
        grow = cbase + p * HCPT + j

        @pl.when(j >= 1)
        def _():
            pltpu.make_async_copy(rbufs[nb], out_sh.at[dst_v.at[0]],
                                  ssems[nb]).wait()

        @pl.when(j + 1 < HCPT)
        def _():
            _issue_gather(j + 1, rbufs[nb], gsems[nb])
            _issue_eatt(grow + 1, ebufs[nb], esems[nb])

        @pl.when(jnp.logical_and(c == 0, j >= 2))
        def _():
            pltpu.make_async_copy(exb, den_sh.at[dst_v.at[0]],
                                  dsems[b]).wait()

        pltpu.make_async_copy(rep_hbm.at[pl.ds(0, L)], ebuf,
                              esems[b]).wait()
        for k in range(K // L):
            sl = pl.ds(k * L, L)
            isrc = src_v[j, sl]
            idst = dst_v[j, sl]
            av = plsc.load_gather(hs_v, [isrc])
            bv = plsc.load_gather(hd_v, [idst])
            shv = plsc.load_gather(sh_v, [idst])
            eattv = plsc.load_gather(ebuf, [rpat + 2 * k, cpat])
            al = av + bv + eattv
            al = jnp.where(al >= 0.0, al, 0.2 * al)
            exb[sl] = jnp.exp(jnp.minimum(al - shv, 80.0))

        @pl.when(c == 0)
        def _():
            pltpu.async_copy(exb, den_sh.at[dst_v.at[j]], dsems[b], add=True)

        pltpu.make_async_copy(h0_hbm.at[src_v.at[0]], buf, gsems[b]).wait()

        for v in range(K // L):
            exv = exb[pl.ds(v * L, L)]
            base = v * L
            for e in range(L):
                sv = exv[e]
                for k in range(DH // L):
                    sl = pl.ds(k * L, L)
                    buf[base + e, sl] = buf[base + e, sl] * sv
        pltpu.async_copy(buf, out_sh.at[dst_v.at[j]], ssems[b], add=True)

    for p in range(NPASS):
        pbase = cbase + p * HCPT
        pltpu.sync_copy(src_hbm.at[pl.ds(pbase, HCPT)], src_v)
        pltpu.sync_copy(dst_hbm.at[pl.ds(pbase, HCPT)], dst_v)
        _issue_gather(0, rows0, gsem0)
        _issue_eatt(pbase, eb0, esem0)

        def _pair(i, _, p=p):
            _chunk(p, i * 2, 0)
            _chunk(p, i * 2 + 1, 1)
            return 0

        lax.fori_loop(0, HCPT // 2, _pair, 0)
        pltpu.make_async_copy(rows1, out_sh.at[dst_v.at[0]], ssem1).wait()

        @pl.when(c == 0)
        def _():
            pltpu.make_async_copy(ex0, den_sh.at[dst_v.at[0]], dsem0).wait()
            pltpu.make_async_copy(ex1, den_sh.at[dst_v.at[0]], dsem1).wait()

    plsc.subcore_barrier()

    pltpu.sync_copy(out_sh.at[pl.ds(rbase, ROWS_PER_TILE)],
                    out_hbm.at[c, pl.ds(rbase, ROWS_PER_TILE)])

    @pl.when(c == 0)
    def _():
        pltpu.sync_copy(den_sh.at[pl.ds(rbase, ROWS_PER_TILE)],
                        den_hbm.at[pl.ds(rbase, ROWS_PER_TILE)])


_sc_layer = pl.kernel(
    _sc_layer_body,
    out_type=[
        jax.ShapeDtypeStruct((NC, N_PAD, DH), jnp.float32),
        jax.ShapeDtypeStruct((N_PAD,), jnp.float32),
    ],
    mesh=plsc.VectorSubcoreMesh(core_axis_name="c", subcore_axis_name="s"),
    compiler_params=pltpu.CompilerParams(
        use_tc_tiling_on_sc=False, needs_layout_passes=False),
    scratch_types=[
        pltpu.VMEM((N_PAD,), jnp.float32),
        pltpu.VMEM((N_PAD,), jnp.float32),
        pltpu.VMEM((N_PAD,), jnp.float32),
        pltpu.VMEM((L, D), jnp.float32),
        pltpu.VMEM((L, D), jnp.float32),
        pltpu.VMEM((HCPT, K), jnp.int32),
        pltpu.VMEM((HCPT, K), jnp.int32),
        pltpu.VMEM((K,), jnp.float32),
        pltpu.VMEM((K,), jnp.float32),
        pltpu.VMEM((K, DH), jnp.float32),
        pltpu.VMEM((K, DH), jnp.float32),
        pltpu.VMEM_SHARED((N_PAD, DH), jnp.float32),
        pltpu.VMEM_SHARED((N_PAD,), jnp.float32),
        pltpu.SemaphoreType.DMA,
        pltpu.SemaphoreType.DMA,
        pltpu.SemaphoreType.DMA,
        pltpu.SemaphoreType.DMA,
        pltpu.SemaphoreType.DMA,
        pltpu.SemaphoreType.DMA,
        pltpu.SemaphoreType.DMA,
        pltpu.SemaphoreType.DMA,
    ],
)


def _epilogue_pool_body(p0_ref, p1_ref, h0_ref, h1_ref, hs_ref, hd_ref,
                        sh_ref, csum_ref, d_ref, b_ref, bid_ref,
                        pooled_ref, cnt_ref):
    i = pl.program_id(0)

    @pl.when(i == 0)
    def _():
        pooled_ref[...] = jnp.zeros_like(pooled_ref)
        cnt_ref[...] = jnp.zeros_like(cnt_ref)

    y = _layer_out(p0_ref, p1_ref, h0_ref, h1_ref, hs_ref, hd_ref, sh_ref,
                   csum_ref, d_ref, b_ref)
    bid = bid_ref[...]
    ones = jnp.ones_like(y)
    for g in range(G):
        m = bid == g
        pooled_ref[g:g + 1, :] += jnp.sum(jnp.where(m, y, 0.0), axis=0,
                                          keepdims=True)
        cnt_ref[g:g + 1, :] += jnp.sum(jnp.where(m, ones, 0.0), axis=0,
                                       keepdims=True)

    @pl.when(i == pl.num_programs(0) - 1)
    def _():
        pooled_ref[...] = pooled_ref[...] / jnp.maximum(cnt_ref[...], 1.0)


def _epilogue_pool(p0, p1, h0, h1, hs, hd, sh, csum, d, b2d, bid2d):
    grid = N_PAD // NB
    return pl.pallas_call(
        _epilogue_pool_body,
        grid=(grid,),
        in_specs=_LAYER_IN_SPECS + [
            pl.BlockSpec((NB, 1), lambda i: (i, 0)),
        ],
        out_specs=pl.BlockSpec((G, D), lambda i: (0, 0)),
        out_shape=jax.ShapeDtypeStruct((G, D), jnp.float32),
        scratch_shapes=[pltpu.VMEM((G, D), jnp.float32)],
    )(p0, p1, h0, h1, hs, hd, sh, csum, d, b2d, bid2d)


def kernel(x, edge_index, edge_attr, batch,
           W1, as1, ad1, We1, ae1, b1, W2, as2, ad2, We2, ae2, b2):
    f32 = jnp.float32
    xp = jnp.zeros((N_PAD, D), f32).at[:N].set(x)
    pad_idx = jnp.full((E_PAD - E,), N_PAD - 1, jnp.int32)
    src2d = jnp.concatenate([edge_index[0], pad_idx]).reshape(E_PAD // K, K)
    dst2d = jnp.concatenate([edge_index[1], pad_idx]).reshape(E_PAD // K, K)
    bid2d = jnp.full((N_PAD, 1), G, jnp.int32).at[:N, 0].set(batch)

    ea2 = edge_attr.reshape(E8, D)

    rep1, csum1 = _eatt_dense(ea2, We1.T, ae1.reshape(1, D))
    h0, h1, hs, hd, sh = _node_dense(xp, W1, as1.reshape(D, 1),
                                     ad1.reshape(D, 1))
    parts1, den1 = _sc_layer(hs.reshape(N_PAD), hd.reshape(N_PAD),
                             sh.reshape(N_PAD), rep1,
                             src2d, dst2d, h0, h1)
    rep2, csum2 = _eatt_dense(ea2, We2.T, ae2.reshape(1, D))
    g0, g1, hs2, hd2, sh2 = _mid_dense(parts1[0], parts1[1], h0, h1,
                                       hs, hd, sh, csum1,
                                       den1.reshape(N_PAD, 1),
                                       b1.reshape(1, D), W2,
                                       as2.reshape(D, 1), ad2.reshape(D, 1))
    parts2, den2 = _sc_layer(hs2.reshape(N_PAD), hd2.reshape(N_PAD),
                             sh2.reshape(N_PAD), rep2,
                             src2d, dst2d, g0, g1)
    pooled = _epilogue_pool(parts2[0], parts2[1], g0, g1,
                            hs2, hd2, sh2, csum2,
                            den2.reshape(N_PAD, 1), b2.reshape(1, D), bid2d)
    return pooled


# speedup vs baseline: 1.6146x; 1.0079x over previous
import jax
import jax.numpy as jnp
from jax import lax
from jax.experimental import pallas as pl
from jax.experimental.pallas import tpu as pltpu
from jax.experimental.pallas import tpu_sc as plsc

N = 10000
E = 320000
D = 128
DE = 16
G = 16

NC = 2
NS = 16
L = 16

N_PAD = 10240
E_PAD = 327680
EPT = E_PAD // NS
K = 128
CPT = EPT // K
NPASS = 2
HCPT = CPT // NPASS
DH = D // NC
ROWS_PER_TILE = N_PAD // NS

NB = 256
EB = 2000


E8 = E // 8
EBR = 1000


def _eatt_dense_body(ea2_ref, wet_ref, aet_ref, rep_ref, csum_ref):
    i = pl.program_id(0)

    @pl.when(i == 0)
    def _():
        csum_ref[...] = jnp.zeros_like(csum_ref)

    w_t = aet_ref[...] @ wet_ref[...]
    wtile = jnp.concatenate([w_t] * 8, axis=1)
    prod = ea2_ref[...] * wtile
    ii = lax.broadcasted_iota(jnp.int32, (D, D), 0) // DE
    jj = lax.broadcasted_iota(jnp.int32, (D, D), 1) // DE
    bo = jnp.where(ii == jj, 1.0, 0.0)
    rep_ref[...] = prod @ bo
    csum_ref[...] += jnp.sum(prod, axis=0, keepdims=True)


def _eatt_dense(ea2, wet, aet):
    grid = E8 // EBR
    return pl.pallas_call(
        _eatt_dense_body,
        grid=(grid,),
        in_specs=[
            pl.BlockSpec((EBR, D), lambda i: (i, 0)),
            pl.BlockSpec((D, DE), lambda i: (0, 0)),
            pl.BlockSpec((1, D), lambda i: (0, 0)),
        ],
        out_specs=[
            pl.BlockSpec((EBR, D), lambda i: (i, 0)),
            pl.BlockSpec((1, D), lambda i: (0, 0)),
        ],
        out_shape=[
            jax.ShapeDtypeStruct((E8, D), jnp.float32),
            jax.ShapeDtypeStruct((1, D), jnp.float32),
        ],
    )(ea2, wet, aet)


def _node_tail(h, as_ref, ad_ref, h0_ref, h1_ref, hs_ref, hd_ref, sh_ref):
    h0_ref[...] = h[:, :DH]
    h1_ref[...] = h[:, DH:]
    hs = h @ as_ref[...]
    hd = h @ ad_ref[...]
    t = hs + hd
    hs_ref[...] = hs
    hd_ref[...] = hd
    sh_ref[...] = jnp.where(t >= 0.0, t, 0.2 * t)


_NODE_OUT_SPECS = [
    pl.BlockSpec((NB, DH), lambda i: (i, 0)),
    pl.BlockSpec((NB, DH), lambda i: (i, 0)),
    pl.BlockSpec((NB, 1), lambda i: (i, 0)),
    pl.BlockSpec((NB, 1), lambda i: (i, 0)),
    pl.BlockSpec((NB, 1), lambda i: (i, 0)),
]

_NODE_OUT_SHAPE = [
    jax.ShapeDtypeStruct((N_PAD, DH), jnp.float32),
    jax.ShapeDtypeStruct((N_PAD, DH), jnp.float32),
    jax.ShapeDtypeStruct((N_PAD, 1), jnp.float32),
    jax.ShapeDtypeStruct((N_PAD, 1), jnp.float32),
    jax.ShapeDtypeStruct((N_PAD, 1), jnp.float32),
]


def _node_dense_body(x_ref, w_ref, as_ref, ad_ref,
                     h0_ref, h1_ref, hs_ref, hd_ref, sh_ref):
    h = x_ref[...] @ w_ref[...]
    _node_tail(h, as_ref, ad_ref, h0_ref, h1_ref, hs_ref, hd_ref, sh_ref)


def _node_dense(xp, w, asv, adv):
    grid = N_PAD // NB
    return pl.pallas_call(
        _node_dense_body,
        grid=(grid,),
        in_specs=[
            pl.BlockSpec((NB, D), lambda i: (i, 0)),
            pl.BlockSpec((D, D), lambda i: (0, 0)),
            pl.BlockSpec((D, 1), lambda i: (0, 0)),
            pl.BlockSpec((D, 1), lambda i: (0, 0)),
        ],
        out_specs=_NODE_OUT_SPECS,
        out_shape=_NODE_OUT_SHAPE,
    )(xp, w, asv, adv)


def _layer_out(p0_ref, p1_ref, h0_ref, h1_ref, hs_ref, hd_ref, sh_ref,
               csum_ref, d_ref, b_ref):
    c = jnp.sum(csum_ref[...]) * (1.0 / E)
    t = hs_ref[...] + hd_ref[...] + c
    t = jnp.where(t >= 0.0, t, 0.2 * t)
    selfex = jnp.exp(jnp.minimum(t - sh_ref[...], 80.0))
    den = d_ref[...] + selfex
    msg = jnp.concatenate(
        [p0_ref[...] + h0_ref[...] * selfex,
         p1_ref[...] + h1_ref[...] * selfex], axis=-1)
    y = msg / den + b_ref[...]
    return y * (1.0 / (1.0 + jnp.exp(-y)))


_LAYER_IN_SPECS = [
    pl.BlockSpec((NB, DH), lambda i: (i, 0)),
    pl.BlockSpec((NB, DH), lambda i: (i, 0)),
    pl.BlockSpec((NB, DH), lambda i: (i, 0)),
    pl.BlockSpec((NB, DH), lambda i: (i, 0)),
    pl.BlockSpec((NB, 1), lambda i: (i, 0)),
    pl.BlockSpec((NB, 1), lambda i: (i, 0)),
    pl.BlockSpec((NB, 1), lambda i: (i, 0)),
    pl.BlockSpec((1, D), lambda i: (0, 0)),
    pl.BlockSpec((NB, 1), lambda i: (i, 0)),
    pl.BlockSpec((1, D), lambda i: (0, 0)),
]


def _mid_dense_body(p0_ref, p1_ref, h0_ref, h1_ref, hs_ref, hd_ref, sh_ref,
                    csum_ref, d_ref, b_ref, w_ref, as_ref, ad_ref,
                    h0o_ref, h1o_ref, hso_ref, hdo_ref, sho_ref):
    y = _layer_out(p0_ref, p1_ref, h0_ref, h1_ref, hs_ref, hd_ref, sh_ref,
                   csum_ref, d_ref, b_ref)
    h = y @ w_ref[...]
    _node_tail(h, as_ref, ad_ref, h0o_ref, h1o_ref, hso_ref, hdo_ref, sho_ref)


def _mid_dense(p0, p1, h0, h1, hs, hd, sh, csum, d, b2d, w, asv, adv):
    grid = N_PAD // NB
    return pl.pallas_call(
        _mid_dense_body,
        grid=(grid,),
        in_specs=_LAYER_IN_SPECS + [
            pl.BlockSpec((D, D), lambda i: (0, 0)),
            pl.BlockSpec((D, 1), lambda i: (0, 0)),
            pl.BlockSpec((D, 1), lambda i: (0, 0)),
        ],
        out_specs=_NODE_OUT_SPECS,
        out_shape=_NODE_OUT_SHAPE,
    )(p0, p1, h0, h1, hs, hd, sh, csum, d, b2d, w, asv, adv)


def _sc_layer_body(hs_hbm, hd_hbm, sh_hbm, rep_hbm, src_hbm, dst_hbm,
                   h0_hbm, h1_hbm, out_hbm, den_hbm,
                   hs_v, hd_v, sh_v, eb0, eb1, src_v, dst_v,
                   ex0, ex1, rows0, rows1, rows2, rows3,
                   out_sh, den_sh,
                   gsem0, gsem1, gsem2, gsem3, ssem0, ssem1, ssem2, ssem3,
                   dsem0, dsem1, esem0, esem1):
    c = lax.axis_index("c")
    s = lax.axis_index("s")
    cbase = s * CPT
    rbufs = (rows0, rows1, rows2, rows3)
    ebufs = (eb0, eb1)
    exbufs = (ex0, ex1)
    gsems = (gsem0, gsem1, gsem2, gsem3)
    ssems = (ssem0, ssem1, ssem2, ssem3)
    dsems = (dsem0, dsem1)
    esems = (esem0, esem1)
    zi = lax.iota(jnp.int32, L)
    rpat = lax.shift_right_logical(zi, 3)
    cpat = (zi - rpat * 8) * L

    pltpu.sync_copy(hs_hbm, hs_v)
    pltpu.sync_copy(hd_hbm, hd_v)
    pltpu.sync_copy(sh_hbm, sh_v)

    def _zrow(i, _):
        for k in range(DH // L):
            rows0[i, pl.ds(k * L, L)] = jnp.zeros((L,), jnp.float32)
        return 0

    lax.fori_loop(0, K, _zrow, 0)
    rbase = s * ROWS_PER_TILE
    for t in range(ROWS_PER_TILE // K):
        pltpu.sync_copy(rows0, out_sh.at[pl.ds(rbase + t * K, K)])

    @pl.when(c == 0)
    def _():
        for t in range(ROWS_PER_TILE // DH):
            pltpu.sync_copy(rows0.at[0],
                            den_sh.at[pl.ds(rbase + t * DH, DH)])

    plsc.subcore_barrier()

    def _issue_gather(j, buf, sem):
        @pl.when(c == 0)
        def _():
            pltpu.async_copy(h0_hbm.at[src_v.at[j]], buf, sem)

        @pl.when(c == 1)
        def _():
            pltpu.async_copy(h1_hbm.at[src_v.at[j]], buf, sem)

    def _issue_eatt(grow, buf, sem):
        growc = jnp.minimum(grow, E // K - 1)
        pltpu.async_copy(rep_hbm.at[pl.ds(growc * L, L)], buf, sem)

    def _chunk(p, j, b4, b2):
        buf = rbufs[b4]
        n4 = (b4 + 2) % 4
        ebuf = ebufs[b2]
        exb = exbufs[b2]
        grow = cbase + p * HCPT + j

        @pl.when(j >= 2)
        def _():
            pltpu.make_async_copy(rbufs[n4], out_sh.at[dst_v.at[0]],
                                  ssems[n4]).wait()

        @pl.when(j + 2 < HCPT)
        def _():
            _issue_gather(j + 2, rbufs[n4], gsems[n4])

        @pl.when(jnp.logical_and(c == 0, j >= 2))
        def _():
            pltpu.make_async_copy(exb, den_sh.at[dst_v.at[0]],
                                  dsems[b2]).wait()

        pltpu.make_async_copy(rep_hbm.at[pl.ds(0, L)], ebuf,
                              esems[b2]).wait()
        for k in range(K // L):
            sl = pl.ds(k * L, L)
            isrc = src_v[j, sl]
            idst = dst_v[j, sl]
            av = plsc.load_gather(hs_v, [isrc])
            bv = plsc.load_gather(hd_v, [idst])
            shv = plsc.load_gather(sh_v, [idst])
            eattv = plsc.load_gather(ebuf, [rpat + 2 * k, cpat])
            al = av + bv + eattv
            al = jnp.where(al >= 0.0, al, 0.2 * al)
            exb[sl] = jnp.exp(jnp.minimum(al - shv, 80.0))

        @pl.when(j + 2 < HCPT)
        def _():
            _issue_eatt(grow + 2, ebuf, esems[b2])

        @pl.when(c == 0)
        def _():
            pltpu.async_copy(exb, den_sh.at[dst_v.at[j]], dsems[b2], add=True)

        pltpu.make_async_copy(h0_hbm.at[src_v.at[0]], buf, gsems[b4]).wait()

        for v in range(K // L):
            exv = exb[pl.ds(v * L, L)]
            base = v * L
            for e in range(L):
                sv = exv[e]
                for k in range(DH // L):
                    sl = pl.ds(k * L, L)
                    buf[base + e, sl] = buf[base + e, sl] * sv
        pltpu.async_copy(buf, out_sh.at[dst_v.at[j]], ssems[b4], add=True)

    for p in range(NPASS):
        pbase = cbase + p * HCPT
        pltpu.sync_copy(src_hbm.at[pl.ds(pbase, HCPT)], src_v)
        pltpu.sync_copy(dst_hbm.at[pl.ds(pbase, HCPT)], dst_v)
        _issue_gather(0, rows0, gsem0)
        _issue_gather(1, rows1, gsem1)
        _issue_eatt(pbase, eb0, esem0)
        _issue_eatt(pbase + 1, eb1, esem1)

        def _quad(i, _, p=p):
            _chunk(p, i * 4, 0, 0)
            _chunk(p, i * 4 + 1, 1, 1)
            _chunk(p, i * 4 + 2, 2, 0)
            _chunk(p, i * 4 + 3, 3, 1)
            return 0

        lax.fori_loop(0, HCPT // 4, _quad, 0)
        pltpu.make_async_copy(rows2, out_sh.at[dst_v.at[0]], ssem2).wait()
        pltpu.make_async_copy(rows3, out_sh.at[dst_v.at[0]], ssem3).wait()

        @pl.when(c == 0)
        def _():
            pltpu.make_async_copy(ex0, den_sh.at[dst_v.at[0]], dsem0).wait()
            pltpu.make_async_copy(ex1, den_sh.at[dst_v.at[0]], dsem1).wait()

    plsc.subcore_barrier()

    pltpu.sync_copy(out_sh.at[pl.ds(rbase, ROWS_PER_TILE)],
                    out_hbm.at[c, pl.ds(rbase, ROWS_PER_TILE)])

    @pl.when(c == 0)
    def _():
        pltpu.sync_copy(den_sh.at[pl.ds(rbase, ROWS_PER_TILE)],
                        den_hbm.at[pl.ds(rbase, ROWS_PER_TILE)])


_sc_layer = pl.kernel(
    _sc_layer_body,
    out_type=[
        jax.ShapeDtypeStruct((NC, N_PAD, DH), jnp.float32),
        jax.ShapeDtypeStruct((N_PAD,), jnp.float32),
    ],
    mesh=plsc.VectorSubcoreMesh(core_axis_name="c", subcore_axis_name="s"),
    compiler_params=pltpu.CompilerParams(
        use_tc_tiling_on_sc=False, needs_layout_passes=False),
    scratch_types=[
        pltpu.VMEM((N_PAD,), jnp.float32),
        pltpu.VMEM((N_PAD,), jnp.float32),
        pltpu.VMEM((N_PAD,), jnp.float32),
        pltpu.VMEM((L, D), jnp.float32),
        pltpu.VMEM((L, D), jnp.float32),
        pltpu.VMEM((HCPT, K), jnp.int32),
        pltpu.VMEM((HCPT, K), jnp.int32),
        pltpu.VMEM((K,), jnp.float32),
        pltpu.VMEM((K,), jnp.float32),
        pltpu.VMEM((K, DH), jnp.float32),
        pltpu.VMEM((K, DH), jnp.float32),
        pltpu.VMEM((K, DH), jnp.float32),
        pltpu.VMEM((K, DH), jnp.float32),
        pltpu.VMEM_SHARED((N_PAD, DH), jnp.float32),
        pltpu.VMEM_SHARED((N_PAD,), jnp.float32),
        pltpu.SemaphoreType.DMA,
        pltpu.SemaphoreType.DMA,
        pltpu.SemaphoreType.DMA,
        pltpu.SemaphoreType.DMA,
        pltpu.SemaphoreType.DMA,
        pltpu.SemaphoreType.DMA,
        pltpu.SemaphoreType.DMA,
        pltpu.SemaphoreType.DMA,
        pltpu.SemaphoreType.DMA,
        pltpu.SemaphoreType.DMA,
        pltpu.SemaphoreType.DMA,
        pltpu.SemaphoreType.DMA,
    ],
)


def _epilogue_pool_body(p0_ref, p1_ref, h0_ref, h1_ref, hs_ref, hd_ref,
                        sh_ref, csum_ref, d_ref, b_ref, bid_ref,
                        pooled_ref, cnt_ref):
    i = pl.program_id(0)

    @pl.when(i == 0)
    def _():
        pooled_ref[...] = jnp.zeros_like(pooled_ref)
        cnt_ref[...] = jnp.zeros_like(cnt_ref)

    y = _layer_out(p0_ref, p1_ref, h0_ref, h1_ref, hs_ref, hd_ref, sh_ref,
                   csum_ref, d_ref, b_ref)
    bid = bid_ref[...]
    ones = jnp.ones_like(y)
    for g in range(G):
        m = bid == g
        pooled_ref[g:g + 1, :] += jnp.sum(jnp.where(m, y, 0.0), axis=0,
                                          keepdims=True)
        cnt_ref[g:g + 1, :] += jnp.sum(jnp.where(m, ones, 0.0), axis=0,
                                       keepdims=True)

    @pl.when(i == pl.num_programs(0) - 1)
    def _():
        pooled_ref[...] = pooled_ref[...] / jnp.maximum(cnt_ref[...], 1.0)


def _epilogue_pool(p0, p1, h0, h1, hs, hd, sh, csum, d, b2d, bid2d):
    grid = N_PAD // NB
    return pl.pallas_call(
        _epilogue_pool_body,
        grid=(grid,),
        in_specs=_LAYER_IN_SPECS + [
            pl.BlockSpec((NB, 1), lambda i: (i, 0)),
        ],
        out_specs=pl.BlockSpec((G, D), lambda i: (0, 0)),
        out_shape=jax.ShapeDtypeStruct((G, D), jnp.float32),
        scratch_shapes=[pltpu.VMEM((G, D), jnp.float32)],
    )(p0, p1, h0, h1, hs, hd, sh, csum, d, b2d, bid2d)


def kernel(x, edge_index, edge_attr, batch,
           W1, as1, ad1, We1, ae1, b1, W2, as2, ad2, We2, ae2, b2):
    f32 = jnp.float32
    xp = jnp.zeros((N_PAD, D), f32).at[:N].set(x)
    pad_idx = jnp.full((E_PAD - E,), N_PAD - 1, jnp.int32)
    src2d = jnp.concatenate([edge_index[0], pad_idx]).reshape(E_PAD // K, K)
    dst2d = jnp.concatenate([edge_index[1], pad_idx]).reshape(E_PAD // K, K)
    bid2d = jnp.full((N_PAD, 1), G, jnp.int32).at[:N, 0].set(batch)

    ea2 = edge_attr.reshape(E8, D)

    rep1, csum1 = _eatt_dense(ea2, We1.T, ae1.reshape(1, D))
    h0, h1, hs, hd, sh = _node_dense(xp, W1, as1.reshape(D, 1),
                                     ad1.reshape(D, 1))
    parts1, den1 = _sc_layer(hs.reshape(N_PAD), hd.reshape(N_PAD),
                             sh.reshape(N_PAD), rep1,
                             src2d, dst2d, h0, h1)
    rep2, csum2 = _eatt_dense(ea2, We2.T, ae2.reshape(1, D))
    g0, g1, hs2, hd2, sh2 = _mid_dense(parts1[0], parts1[1], h0, h1,
                                       hs, hd, sh, csum1,
                                       den1.reshape(N_PAD, 1),
                                       b1.reshape(1, D), W2,
                                       as2.reshape(D, 1), ad2.reshape(D, 1))
    parts2, den2 = _sc_layer(hs2.reshape(N_PAD), hd2.reshape(N_PAD),
                             sh2.reshape(N_PAD), rep2,
                             src2d, dst2d, g0, g1)
    pooled = _epilogue_pool(parts2[0], parts2[1], g0, g1,
                            hs2, hd2, sh2, csum2,
                            den2.reshape(N_PAD, 1), b2.reshape(1, D), bid2d)
    return pooled
